# baseline (speedup 1.0000x reference)
import functools

import jax
import jax.numpy as jnp
from jax import lax
from jax.experimental import pallas as pl
from jax.experimental.pallas import tpu as pltpu
from jax.experimental.pallas import tpu_sc as plsc

_B, _C, _N = 2, 16, 8192
_DS = 4
_M = _N // _DS
_K = 32
_XYZN = 7
_EPS = 1e-5
_QT = 128
_ST = 2048
_MT = 256
_TOTAL = _B * _K * _M


def _knn_kern(pts_ref, q_ref, out_ref, d_ref):
    px = pts_ref[0, 0:1, :]
    py = pts_ref[0, 1:2, :]
    pz = pts_ref[0, 2:3, :]
    qx = q_ref[0, :, 0:1]
    qy = q_ref[0, :, 1:2]
    qz = q_ref[0, :, 2:3]
    d_ref[...] = (qx - px) ** 2 + (qy - py) ** 2 + (qz - pz) ** 2
    iota = lax.broadcasted_iota(jnp.int32, (_QT, _N), 1)

    def body(k, am_prev):
        d = jnp.where(iota == am_prev[:, None], jnp.inf, d_ref[...])
        d_ref[...] = d
        am = jnp.argmin(d, axis=1).astype(jnp.int32)
        out_ref[0, pl.ds(k, 1), :] = am[None, :]
        return am

    lax.fori_loop(0, _K, body, jnp.full((_QT,), -1, jnp.int32))


def _knn(pts, qT):
    return pl.pallas_call(
        _knn_kern,
        grid=(_B, _M // _QT),
        in_specs=[
            pl.BlockSpec((1, 3, _N), lambda b, t: (b, 0, 0)),
            pl.BlockSpec((1, _QT, 3), lambda b, t: (b, t, 0)),
        ],
        out_specs=pl.BlockSpec((1, _K, _QT), lambda b, t: (b, 0, t)),
        out_shape=jax.ShapeDtypeStruct((_B, _K, _M), jnp.int32),
        scratch_shapes=[pltpu.VMEM((_QT, _N), jnp.float32)],
    )(pts, qT)


_NW = 32
_CH = 128


def _gather_sc(table, flat_idx):
    per_w = _TOTAL // _NW
    n_ch = per_w // _CH
    info = plsc.get_sparse_core_info()
    nc = info.num_cores
    mesh = plsc.VectorSubcoreMesh(core_axis_name="c", subcore_axis_name="s")

    @functools.partial(
        pl.kernel,
        mesh=mesh,
        compiler_params=pltpu.CompilerParams(use_tc_tiling_on_sc=False),
        out_type=jax.ShapeDtypeStruct((_TOTAL, _C), jnp.float32),
        scratch_types=[
            pltpu.VMEM((_CH,), jnp.int32),
            pltpu.VMEM((_CH, _C), jnp.float32),
            pltpu.SemaphoreType.DMA,
        ],
    )
    def gk(table_hbm, idx_hbm, out_hbm, idx_v, rows_v, sem):
        wid = lax.axis_index("s") * nc + lax.axis_index("c")

        def body(i, _):
            base = wid * per_w + i * _CH
            pltpu.sync_copy(idx_hbm.at[pl.ds(base, _CH)], idx_v)
            pltpu.async_copy(table_hbm.at[idx_v], rows_v, sem).wait()
            pltpu.sync_copy(rows_v, out_hbm.at[pl.ds(base, _CH)])
            return 0

        lax.fori_loop(0, n_ch, body, 0)

    return gk(table, flat_idx)


def _stats1_kern(v_ref, qT_ref, w1aT_ref, w1xT_ref, s1_ref, s2_ref):
    h1 = jnp.dot(v_ref[...], w1aT_ref[...], preferred_element_type=jnp.float32)
    h1 = h1 - jnp.dot(qT_ref[0], w1xT_ref[...], preferred_element_type=jnp.float32)

    @pl.when(pl.program_id(0) == 0)
    def _():
        s1_ref[...] = jnp.zeros_like(s1_ref)
        s2_ref[...] = jnp.zeros_like(s2_ref)

    s1_ref[...] += jnp.sum(h1, axis=0, keepdims=True)
    s2_ref[...] += jnp.sum(h1 * h1, axis=0, keepdims=True)


def _stats1(v, qT, w1aT, w1xT):
    return pl.pallas_call(
        _stats1_kern,
        grid=(_TOTAL // _ST,),
        in_specs=[
            pl.BlockSpec((_ST, _C), lambda s: (s, 0)),
            pl.BlockSpec((1, _M, 3), lambda s: (s // _K, 0, 0)),
            pl.BlockSpec((_C, 64), lambda s: (0, 0)),
            pl.BlockSpec((3, 64), lambda s: (0, 0)),
        ],
        out_specs=[
            pl.BlockSpec((1, 64), lambda s: (0, 0)),
            pl.BlockSpec((1, 64), lambda s: (0, 0)),
        ],
        out_shape=[
            jax.ShapeDtypeStruct((1, 64), jnp.float32),
            jax.ShapeDtypeStruct((1, 64), jnp.float32),
        ],
    )(v, qT, w1aT, w1xT)


def _stats2_kern(v_ref, qT_ref, w1aT_ref, w1xT_ref, s1_ref, s2_ref,
                 g1_ref, b1_ref, w2T_ref, t1_ref, t2_ref):
    h1 = jnp.dot(v_ref[...], w1aT_ref[...], preferred_element_type=jnp.float32)
    h1 = h1 - jnp.dot(qT_ref[0], w1xT_ref[...], preferred_element_type=jnp.float32)
    mu1 = s1_ref[...] / _TOTAL
    var1 = s2_ref[...] / _TOTAL - mu1 * mu1
    sc1 = g1_ref[...] * lax.rsqrt(var1 + _EPS)
    h1r = jnp.maximum((h1 - mu1) * sc1 + b1_ref[...], 0.0)
    h2 = jnp.dot(h1r, w2T_ref[...], preferred_element_type=jnp.float32)

    @pl.when(pl.program_id(0) == 0)
    def _():
        t1_ref[...] = jnp.zeros_like(t1_ref)
        t2_ref[...] = jnp.zeros_like(t2_ref)

    t1_ref[...] += jnp.sum(h2, axis=0, keepdims=True)
    t2_ref[...] += jnp.sum(h2 * h2, axis=0, keepdims=True)


def _stats2(v, qT, w1aT, w1xT, s1, s2, g1r, b1r, w2T):
    return pl.pallas_call(
        _stats2_kern,
        grid=(_TOTAL // _ST,),
        in_specs=[
            pl.BlockSpec((_ST, _C), lambda s: (s, 0)),
            pl.BlockSpec((1, _M, 3), lambda s: (s // _K, 0, 0)),
            pl.BlockSpec((_C, 64), lambda s: (0, 0)),
            pl.BlockSpec((3, 64), lambda s: (0, 0)),
            pl.BlockSpec((1, 64), lambda s: (0, 0)),
            pl.BlockSpec((1, 64), lambda s: (0, 0)),
            pl.BlockSpec((1, 64), lambda s: (0, 0)),
            pl.BlockSpec((1, 64), lambda s: (0, 0)),
            pl.BlockSpec((64, 128), lambda s: (0, 0)),
        ],
        out_specs=[
            pl.BlockSpec((1, 128), lambda s: (0, 0)),
            pl.BlockSpec((1, 128), lambda s: (0, 0)),
        ],
        out_shape=[
            jax.ShapeDtypeStruct((1, 128), jnp.float32),
            jax.ShapeDtypeStruct((1, 128), jnp.float32),
        ],
    )(v, qT, w1aT, w1xT, s1, s2, g1r, b1r, w2T)


def _final_kern(v_ref, qT_ref, w1aT_ref, w1xT_ref, s1_ref, s2_ref,
                g1_ref, b1_ref, w2T_ref, t1_ref, t2_ref, g2_ref, b2_ref,
                o_ref):
    v2 = v_ref[0].reshape(_K * _MT, _C)
    h1 = jnp.dot(v2, w1aT_ref[...], preferred_element_type=jnp.float32)
    pt = jnp.dot(qT_ref[0], w1xT_ref[...], preferred_element_type=jnp.float32)
    h1 = (h1.reshape(_K, _MT, 64) - pt[None]).reshape(_K * _MT, 64)
    mu1 = s1_ref[...] / _TOTAL
    var1 = s2_ref[...] / _TOTAL - mu1 * mu1
    sc1 = g1_ref[...] * lax.rsqrt(var1 + _EPS)
    h1r = jnp.maximum((h1 - mu1) * sc1 + b1_ref[...], 0.0)
    h2 = jnp.dot(h1r, w2T_ref[...], preferred_element_type=jnp.float32)
    mu2 = t1_ref[...] / _TOTAL
    var2 = t2_ref[...] / _TOTAL - mu2 * mu2
    sc2 = g2_ref[...] * lax.rsqrt(var2 + _EPS)
    h2r = jnp.maximum((h2 - mu2) * sc2 + b2_ref[...], 0.0)
    o_ref[0] = jnp.max(h2r.reshape(_K, _MT, 128), axis=0)


def _final(v4, qT, w1aT, w1xT, s1, s2, g1r, b1r, w2T, t1, t2, g2r, b2r):
    return pl.pallas_call(
        _final_kern,
        grid=(_B, _M // _MT),
        in_specs=[
            pl.BlockSpec((1, _K, _MT, _C), lambda b, t: (b, 0, t, 0)),
            pl.BlockSpec((1, _MT, 3), lambda b, t: (b, t, 0)),
            pl.BlockSpec((_C, 64), lambda b, t: (0, 0)),
            pl.BlockSpec((3, 64), lambda b, t: (0, 0)),
            pl.BlockSpec((1, 64), lambda b, t: (0, 0)),
            pl.BlockSpec((1, 64), lambda b, t: (0, 0)),
            pl.BlockSpec((1, 64), lambda b, t: (0, 0)),
            pl.BlockSpec((1, 64), lambda b, t: (0, 0)),
            pl.BlockSpec((64, 128), lambda b, t: (0, 0)),
            pl.BlockSpec((1, 128), lambda b, t: (0, 0)),
            pl.BlockSpec((1, 128), lambda b, t: (0, 0)),
            pl.BlockSpec((1, 128), lambda b, t: (0, 0)),
            pl.BlockSpec((1, 128), lambda b, t: (0, 0)),
        ],
        out_specs=pl.BlockSpec((1, _MT, 128), lambda b, t: (b, t, 0)),
        out_shape=jax.ShapeDtypeStruct((_B, _M, 128), jnp.float32),
    )(v4, qT, w1aT, w1xT, s1, s2, g1r, b1r, w2T, t1, t2, g2r, b2r)


def kernel(x, W1, g1, b1, W2, g2, b2):
    x3 = x[:, :, :, 0]
    pts = x3[:, 0:3, :]
    qT = jnp.transpose(x3[:, 0:3, ::_DS], (0, 2, 1))

    idx = _knn(pts, qT)

    table = jnp.transpose(x3, (0, 2, 1)).reshape(_B * _N, _C)
    flat_idx = (idx + (jnp.arange(_B, dtype=jnp.int32) * _N)[:, None, None]
                ).reshape(-1)
    v = _gather_sc(table, flat_idx)

    w1a = jnp.concatenate(
        [W1[:, 0:6], jnp.zeros((64, 1), jnp.float32), W1[:, 6:15]], axis=1)
    w1aT = w1a.T
    w1xT = W1[:, 0:3].T
    g1r, b1r = g1.reshape(1, 64), b1.reshape(1, 64)
    g2r, b2r = g2.reshape(1, 128), b2.reshape(1, 128)
    w2T = W2.T

    s1, s2 = _stats1(v, qT, w1aT, w1xT)
    t1, t2 = _stats2(v, qT, w1aT, w1xT, s1, s2, g1r, b1r, w2T)
    o = _final(v.reshape(_B, _K, _M, _C), qT, w1aT, w1xT,
               s1, s2, g1r, b1r, w2T, t1, t2, g2r, b2r)

    pd = x[:, 0:_XYZN, ::_DS, :]
    return jnp.concatenate(
        [pd, jnp.transpose(o, (0, 2, 1))[..., None]], axis=1)


# speedup vs baseline: 1.1819x; 1.1819x over previous
import functools

import jax
import jax.numpy as jnp
from jax import lax
from jax.experimental import pallas as pl
from jax.experimental.pallas import tpu as pltpu
from jax.experimental.pallas import tpu_sc as plsc

_B, _C, _N = 2, 16, 8192
_DS = 4
_M = _N // _DS
_K = 32
_XYZN = 7
_EPS = 1e-5
_QT = 128
_ST = 2048
_MT = 256
_TOTAL = _B * _K * _M
_NW = 32


def _knn_kern(pts_ref, q_ref, out_ref, d_ref):
    px = pts_ref[0, 0:1, :]
    py = pts_ref[0, 1:2, :]
    pz = pts_ref[0, 2:3, :]
    qx = q_ref[0, :, 0:1]
    qy = q_ref[0, :, 1:2]
    qz = q_ref[0, :, 2:3]
    d_ref[...] = (qx - px) ** 2 + (qy - py) ** 2 + (qz - pz) ** 2
    iota = lax.broadcasted_iota(jnp.int32, (_QT, _N), 1)

    def body(k, _):
        d = d_ref[...]
        mn = jnp.min(d, axis=1, keepdims=True)
        am = jnp.min(jnp.where(d == mn, iota, _N), axis=1)
        out_ref[0, pl.ds(k, 1), :] = am[None, :]
        d_ref[...] = jnp.where(iota == am[:, None], jnp.inf, d)
        return 0

    lax.fori_loop(0, _K, body, 0)


def _knn(pts, qT):
    return pl.pallas_call(
        _knn_kern,
        grid=(_B, _M // _QT),
        in_specs=[
            pl.BlockSpec((1, 3, _N), lambda b, t: (b, 0, 0)),
            pl.BlockSpec((1, _QT, 3), lambda b, t: (b, t, 0)),
        ],
        out_specs=pl.BlockSpec((1, _K, _QT), lambda b, t: (b, 0, t)),
        out_shape=jax.ShapeDtypeStruct((_B, _K, _M), jnp.int32),
        scratch_shapes=[pltpu.VMEM((_QT, _N), jnp.float32)],
    )(pts, qT)


_QPW = _M * _B // _NW
_NG = 512
_GV = _NG // 16


def _knn_sc(pts, qprep):
    info = plsc.get_sparse_core_info()
    nc = info.num_cores
    mesh = plsc.VectorSubcoreMesh(core_axis_name="c", subcore_axis_name="s")

    @functools.partial(
        pl.kernel,
        mesh=mesh,
        compiler_params=pltpu.CompilerParams(
            use_tc_tiling_on_sc=False, needs_layout_passes=False),
        out_type=jax.ShapeDtypeStruct((_B * _M * _K,), jnp.int32),
        scratch_types=[
            pltpu.VMEM((3, _N), jnp.float32),
            pltpu.VMEM((3, _QPW), jnp.float32),
            pltpu.VMEM((_N,), jnp.float32),
            pltpu.VMEM((_NG,), jnp.float32),
            pltpu.VMEM((32,), jnp.float32),
            pltpu.VMEM((_QPW * _K,), jnp.int32),
        ],
    )
    def kk(pts_hbm, q_hbm, out_hbm, ptsv, qv, dbuf, gm, gmm, idxout):
        w = lax.axis_index("s") * nc + lax.axis_index("c")
        b = w // (_NW // _B)
        pltpu.sync_copy(pts_hbm.at[b], ptsv)
        pltpu.sync_copy(q_hbm.at[w], qv)

        iota = lax.broadcasted_iota(jnp.int32, (16,), 0)
        lane0 = iota == 0
        zero16 = jnp.zeros((16,), jnp.int32)
        one16 = jnp.full((16,), 1, jnp.int32)
        two16 = jnp.full((16,), 2, jnp.int32)
        inf16 = jnp.full((16,), jnp.inf, jnp.float32)

        def vmin_splat(x):
            nx = -x
            return -plsc.cummax(jnp.flip(plsc.cummax(nx)))

        def per_query(qi, qis):
            qxs = plsc.load_gather(qv, [zero16, qis])
            qys = plsc.load_gather(qv, [one16, qis])
            qzs = plsc.load_gather(qv, [two16, qis])

            def dist_chunk(c):
                px = ptsv[0, pl.ds(c * 16, 16)]
                py = ptsv[1, pl.ds(c * 16, 16)]
                pz = ptsv[2, pl.ds(c * 16, 16)]
                dx = qxs - px
                dy = qys - py
                dz = qzs - pz
                d = dx * dx + dy * dy + dz * dz
                dbuf[pl.ds(c * 16, 16)] = d
                return d

            def init_chunk(c, _):
                gm[pl.ds(c * 16, 16)] = dist_chunk(c)
                return 0

            lax.fori_loop(0, _GV, init_chunk, 0, unroll=4)

            def fold_chunk(c, _):
                d = dist_chunk(c)
                off = (c % _GV) * 16
                gm[pl.ds(off, 16)] = jnp.minimum(gm[pl.ds(off, 16)], d)
                return 0

            lax.fori_loop(_GV, _N // 16, fold_chunk, 0, unroll=4)

            def gmm_fold(j, js):
                s = vmin_splat(gm[pl.ds(j * 16, 16)])
                plsc.store_scatter(gmm, [js], s, mask=lane0)
                return js + 1

            lax.fori_loop(0, _GV, gmm_fold, zero16, unroll=4)

            def extract(i, ks):
                m2a = gmm[pl.ds(0, 16)]
                m2b = gmm[pl.ds(16, 16)]
                mm = jnp.minimum(m2a, m2b)
                hh = jnp.where(m2b < m2a, one16, zero16)
                gmin = vmin_splat(mm)
                l2 = plsc.all_reduce_ffs(mm == gmin)
                hlane = vmin_splat(jnp.where(iota == l2, hh, two16))
                jstar = l2 + 16 * hlane
                gmv = plsc.load_gather(gm, [jstar * 16 + iota])
                lstar = plsc.all_reduce_ffs(gmv == gmin)
                gstar = 16 * jstar + lstar
                dv = plsc.load_gather(dbuf, [iota * _NG + gstar])
                em = vmin_splat(dv)
                el = plsc.all_reduce_ffs(dv == em)
                estar = el * _NG + gstar
                plsc.store_scatter(idxout, [ks], estar, mask=lane0)
                plsc.store_scatter(dbuf, [estar], inf16, mask=lane0)
                ng = vmin_splat(jnp.where(iota == el, inf16, dv))
                plsc.store_scatter(gm, [gstar], ng, mask=lane0)
                ng2 = vmin_splat(jnp.where(iota == lstar, ng, gmv))
                plsc.store_scatter(gmm, [jstar], ng2, mask=lane0)
                return ks + 1

            lax.fori_loop(0, _K, extract, qis * _K)
            return qis + 1

        lax.fori_loop(0, _QPW, per_query, zero16)
        pltpu.sync_copy(idxout, out_hbm.at[pl.ds(w * _QPW * _K, _QPW * _K)])

    return kk(pts, qprep)


_CH = 128


def _gather_sc(table, flat_idx):
    per_w = _TOTAL // _NW
    n_ch = per_w // _CH
    info = plsc.get_sparse_core_info()
    nc = info.num_cores
    mesh = plsc.VectorSubcoreMesh(core_axis_name="c", subcore_axis_name="s")

    @functools.partial(
        pl.kernel,
        mesh=mesh,
        compiler_params=pltpu.CompilerParams(
            use_tc_tiling_on_sc=False, needs_layout_passes=False),
        out_type=jax.ShapeDtypeStruct((_TOTAL, _C), jnp.float32),
        scratch_types=[
            pltpu.VMEM((_CH,), jnp.int32),
            pltpu.VMEM((_CH, _C), jnp.float32),
            pltpu.SemaphoreType.DMA,
        ],
    )
    def gk(table_hbm, idx_hbm, out_hbm, idx_v, rows_v, sem):
        wid = lax.axis_index("s") * nc + lax.axis_index("c")

        def body(i, _):
            base = wid * per_w + i * _CH
            pltpu.sync_copy(idx_hbm.at[pl.ds(base, _CH)], idx_v)
            pltpu.async_copy(table_hbm.at[idx_v], rows_v, sem).wait()
            pltpu.sync_copy(rows_v, out_hbm.at[pl.ds(base, _CH)])
            return 0

        lax.fori_loop(0, n_ch, body, 0)

    return gk(table, flat_idx)


_SQ = _ST // _K


def _stats1_kern(v_ref, qT_ref, w1aT_ref, w1xT_ref, s1_ref, s2_ref):
    h1 = jnp.dot(v_ref[...], w1aT_ref[...], preferred_element_type=jnp.float32)
    pt = jnp.dot(qT_ref[0], w1xT_ref[...], preferred_element_type=jnp.float32)
    h1 = (h1.reshape(_SQ, _K, 64) - pt[:, None, :]).reshape(_ST, 64)

    @pl.when(pl.program_id(0) == 0)
    def _():
        s1_ref[...] = jnp.zeros_like(s1_ref)
        s2_ref[...] = jnp.zeros_like(s2_ref)

    s1_ref[...] += jnp.sum(h1, axis=0, keepdims=True)
    s2_ref[...] += jnp.sum(h1 * h1, axis=0, keepdims=True)


def _stats1(v, qT, w1aT, w1xT):
    return pl.pallas_call(
        _stats1_kern,
        grid=(_TOTAL // _ST,),
        in_specs=[
            pl.BlockSpec((_ST, _C), lambda s: (s, 0)),
            pl.BlockSpec((1, _SQ, 3), lambda s: (s // (_M // _SQ), s % (_M // _SQ), 0)),
            pl.BlockSpec((_C, 64), lambda s: (0, 0)),
            pl.BlockSpec((3, 64), lambda s: (0, 0)),
        ],
        out_specs=[
            pl.BlockSpec((1, 64), lambda s: (0, 0)),
            pl.BlockSpec((1, 64), lambda s: (0, 0)),
        ],
        out_shape=[
            jax.ShapeDtypeStruct((1, 64), jnp.float32),
            jax.ShapeDtypeStruct((1, 64), jnp.float32),
        ],
    )(v, qT, w1aT, w1xT)


def _stats2_kern(v_ref, qT_ref, w1aT_ref, w1xT_ref, s1_ref, s2_ref,
                 g1_ref, b1_ref, w2T_ref, t1_ref, t2_ref):
    h1 = jnp.dot(v_ref[...], w1aT_ref[...], preferred_element_type=jnp.float32)
    pt = jnp.dot(qT_ref[0], w1xT_ref[...], preferred_element_type=jnp.float32)
    h1 = (h1.reshape(_SQ, _K, 64) - pt[:, None, :]).reshape(_ST, 64)
    mu1 = s1_ref[...] / _TOTAL
    var1 = s2_ref[...] / _TOTAL - mu1 * mu1
    sc1 = g1_ref[...] * lax.rsqrt(var1 + _EPS)
    h1r = jnp.maximum((h1 - mu1) * sc1 + b1_ref[...], 0.0)
    h2 = jnp.dot(h1r, w2T_ref[...], preferred_element_type=jnp.float32)

    @pl.when(pl.program_id(0) == 0)
    def _():
        t1_ref[...] = jnp.zeros_like(t1_ref)
        t2_ref[...] = jnp.zeros_like(t2_ref)

    t1_ref[...] += jnp.sum(h2, axis=0, keepdims=True)
    t2_ref[...] += jnp.sum(h2 * h2, axis=0, keepdims=True)


def _stats2(v, qT, w1aT, w1xT, s1, s2, g1r, b1r, w2T):
    return pl.pallas_call(
        _stats2_kern,
        grid=(_TOTAL // _ST,),
        in_specs=[
            pl.BlockSpec((_ST, _C), lambda s: (s, 0)),
            pl.BlockSpec((1, _SQ, 3), lambda s: (s // (_M // _SQ), s % (_M // _SQ), 0)),
            pl.BlockSpec((_C, 64), lambda s: (0, 0)),
            pl.BlockSpec((3, 64), lambda s: (0, 0)),
            pl.BlockSpec((1, 64), lambda s: (0, 0)),
            pl.BlockSpec((1, 64), lambda s: (0, 0)),
            pl.BlockSpec((1, 64), lambda s: (0, 0)),
            pl.BlockSpec((1, 64), lambda s: (0, 0)),
            pl.BlockSpec((64, 128), lambda s: (0, 0)),
        ],
        out_specs=[
            pl.BlockSpec((1, 128), lambda s: (0, 0)),
            pl.BlockSpec((1, 128), lambda s: (0, 0)),
        ],
        out_shape=[
            jax.ShapeDtypeStruct((1, 128), jnp.float32),
            jax.ShapeDtypeStruct((1, 128), jnp.float32),
        ],
    )(v, qT, w1aT, w1xT, s1, s2, g1r, b1r, w2T)


def _final_kern(v_ref, qT_ref, w1aT_ref, w1xT_ref, s1_ref, s2_ref,
                g1_ref, b1_ref, w2T_ref, t1_ref, t2_ref, g2_ref, b2_ref,
                o_ref):
    v2 = v_ref[0].reshape(_MT * _K, _C)
    h1 = jnp.dot(v2, w1aT_ref[...], preferred_element_type=jnp.float32)
    pt = jnp.dot(qT_ref[0], w1xT_ref[...], preferred_element_type=jnp.float32)
    h1 = (h1.reshape(_MT, _K, 64) - pt[:, None, :]).reshape(_MT * _K, 64)
    mu1 = s1_ref[...] / _TOTAL
    var1 = s2_ref[...] / _TOTAL - mu1 * mu1
    sc1 = g1_ref[...] * lax.rsqrt(var1 + _EPS)
    h1r = jnp.maximum((h1 - mu1) * sc1 + b1_ref[...], 0.0)
    h2 = jnp.dot(h1r, w2T_ref[...], preferred_element_type=jnp.float32)
    mu2 = t1_ref[...] / _TOTAL
    var2 = t2_ref[...] / _TOTAL - mu2 * mu2
    sc2 = g2_ref[...] * lax.rsqrt(var2 + _EPS)
    h2r = jnp.maximum((h2 - mu2) * sc2 + b2_ref[...], 0.0)
    o_ref[0] = jnp.max(h2r.reshape(_MT, _K, 128), axis=1)


def _final(v4, qT, w1aT, w1xT, s1, s2, g1r, b1r, w2T, t1, t2, g2r, b2r):
    return pl.pallas_call(
        _final_kern,
        grid=(_B, _M // _MT),
        in_specs=[
            pl.BlockSpec((1, _MT, _K, _C), lambda b, t: (b, t, 0, 0)),
            pl.BlockSpec((1, _MT, 3), lambda b, t: (b, t, 0)),
            pl.BlockSpec((_C, 64), lambda b, t: (0, 0)),
            pl.BlockSpec((3, 64), lambda b, t: (0, 0)),
            pl.BlockSpec((1, 64), lambda b, t: (0, 0)),
            pl.BlockSpec((1, 64), lambda b, t: (0, 0)),
            pl.BlockSpec((1, 64), lambda b, t: (0, 0)),
            pl.BlockSpec((1, 64), lambda b, t: (0, 0)),
            pl.BlockSpec((64, 128), lambda b, t: (0, 0)),
            pl.BlockSpec((1, 128), lambda b, t: (0, 0)),
            pl.BlockSpec((1, 128), lambda b, t: (0, 0)),
            pl.BlockSpec((1, 128), lambda b, t: (0, 0)),
            pl.BlockSpec((1, 128), lambda b, t: (0, 0)),
        ],
        out_specs=pl.BlockSpec((1, _MT, 128), lambda b, t: (b, t, 0)),
        out_shape=jax.ShapeDtypeStruct((_B, _M, 128), jnp.float32),
    )(v4, qT, w1aT, w1xT, s1, s2, g1r, b1r, w2T, t1, t2, g2r, b2r)


def kernel(x, W1, g1, b1, W2, g2, b2):
    x3 = x[:, :, :, 0]
    pts = x3[:, 0:3, :]
    qc = x3[:, 0:3, ::_DS]
    qT = jnp.transpose(qc, (0, 2, 1))
    qprep = (qc.reshape(_B, 3, _NW // _B, _QPW)
             .transpose(0, 2, 1, 3).reshape(_NW, 3, _QPW))

    idx = _knn_sc(pts, qprep)

    table = jnp.transpose(x3, (0, 2, 1)).reshape(_B * _N, _C)
    flat_idx = (idx.reshape(_B, _M * _K)
                + (jnp.arange(_B, dtype=jnp.int32) * _N)[:, None]).reshape(-1)
    v = _gather_sc(table, flat_idx)

    w1a = jnp.concatenate(
        [W1[:, 0:6], jnp.zeros((64, 1), jnp.float32), W1[:, 6:15]], axis=1)
    w1aT = w1a.T
    w1xT = W1[:, 0:3].T
    g1r, b1r = g1.reshape(1, 64), b1.reshape(1, 64)
    g2r, b2r = g2.reshape(1, 128), b2.reshape(1, 128)
    w2T = W2.T

    s1, s2 = _stats1(v, qT, w1aT, w1xT)
    t1, t2 = _stats2(v, qT, w1aT, w1xT, s1, s2, g1r, b1r, w2T)
    o = _final(v.reshape(_B, _M, _K, _C), qT, w1aT, w1xT,
               s1, s2, g1r, b1r, w2T, t1, t2, g2r, b2r)

    pd = x[:, 0:_XYZN, ::_DS, :]
    return jnp.concatenate(
        [pd, jnp.transpose(o, (0, 2, 1))[..., None]], axis=1)


# speedup vs baseline: 2.1411x; 1.8116x over previous
import functools

import jax
import jax.numpy as jnp
from jax import lax
from jax.experimental import pallas as pl
from jax.experimental.pallas import tpu as pltpu
from jax.experimental.pallas import tpu_sc as plsc

_B, _C, _N = 2, 16, 8192
_DS = 4
_M = _N // _DS
_K = 32
_XYZN = 7
_EPS = 1e-5
_QT = 128
_ST = 2048
_MT = 256
_TOTAL = _B * _K * _M
_NW = 32


def _knn_kern(pts_ref, q_ref, out_ref, d_ref):
    px = pts_ref[0, 0:1, :]
    py = pts_ref[0, 1:2, :]
    pz = pts_ref[0, 2:3, :]
    qx = q_ref[0, :, 0:1]
    qy = q_ref[0, :, 1:2]
    qz = q_ref[0, :, 2:3]
    d_ref[...] = (qx - px) ** 2 + (qy - py) ** 2 + (qz - pz) ** 2
    iota = lax.broadcasted_iota(jnp.int32, (_QT, _N), 1)

    def body(k, _):
        d = d_ref[...]
        mn = jnp.min(d, axis=1, keepdims=True)
        am = jnp.min(jnp.where(d == mn, iota, _N), axis=1)
        out_ref[0, pl.ds(k, 1), :] = am[None, :]
        d_ref[...] = jnp.where(iota == am[:, None], jnp.inf, d)
        return 0

    lax.fori_loop(0, _K, body, 0)


def _knn(pts, qT):
    return pl.pallas_call(
        _knn_kern,
        grid=(_B, _M // _QT),
        in_specs=[
            pl.BlockSpec((1, 3, _N), lambda b, t: (b, 0, 0)),
            pl.BlockSpec((1, _QT, 3), lambda b, t: (b, t, 0)),
        ],
        out_specs=pl.BlockSpec((1, _K, _QT), lambda b, t: (b, 0, t)),
        out_shape=jax.ShapeDtypeStruct((_B, _K, _M), jnp.int32),
        scratch_shapes=[pltpu.VMEM((_QT, _N), jnp.float32)],
    )(pts, qT)


_QPW = _M * _B // _NW
_NG = 512
_GV = _NG // 16


def _knn_sc(pts, qprep):
    info = plsc.get_sparse_core_info()
    nc = info.num_cores
    mesh = plsc.VectorSubcoreMesh(core_axis_name="c", subcore_axis_name="s")

    @functools.partial(
        pl.kernel,
        mesh=mesh,
        compiler_params=pltpu.CompilerParams(
            use_tc_tiling_on_sc=False, needs_layout_passes=False),
        out_type=jax.ShapeDtypeStruct((_B * _M * _K,), jnp.int32),
        scratch_types=[
            pltpu.VMEM((3, _N), jnp.float32),
            pltpu.VMEM((3, _QPW), jnp.float32),
            pltpu.VMEM((_N,), jnp.float32),
            pltpu.VMEM((_NG,), jnp.float32),
            pltpu.VMEM((32,), jnp.float32),
            pltpu.VMEM((_QPW * _K,), jnp.int32),
        ],
    )
    def kk(pts_hbm, q_hbm, out_hbm, ptsv, qv, dbuf, gm, gmm, idxout):
        w = lax.axis_index("s") * nc + lax.axis_index("c")
        b = w // (_NW // _B)
        pltpu.sync_copy(pts_hbm.at[b], ptsv)
        pltpu.sync_copy(q_hbm.at[w], qv)

        iota = lax.broadcasted_iota(jnp.int32, (16,), 0)
        lane0 = iota == 0
        lane1 = iota == 1
        zero16 = jnp.zeros((16,), jnp.int32)
        one16 = jnp.full((16,), 1, jnp.int32)
        two16 = jnp.full((16,), 2, jnp.int32)
        sixteen16 = jnp.full((16,), 16, jnp.int32)
        inf16 = jnp.full((16,), jnp.inf, jnp.float32)
        iota16x = iota * 16
        iota512 = iota * _NG

        def vmin_splat(x):
            nx = -x
            return -plsc.cummax(jnp.flip(plsc.cummax(nx)))

        def per_query(qi, qis):
            qxs = plsc.load_gather(qv, [zero16, qis])
            qys = plsc.load_gather(qv, [one16, qis])
            qzs = plsc.load_gather(qv, [two16, qis])

            def dist_chunk(c):
                px = ptsv[0, pl.ds(c * 16, 16)]
                py = ptsv[1, pl.ds(c * 16, 16)]
                pz = ptsv[2, pl.ds(c * 16, 16)]
                dx = qxs - px
                dy = qys - py
                dz = qzs - pz
                d = dx * dx + dy * dy + dz * dz
                dbuf[pl.ds(c * 16, 16)] = d
                return d

            def outer(j, vh):
                def inner(k, acc):
                    return jnp.minimum(acc, dist_chunk(j + _GV * k))

                acc = lax.fori_loop(1, 16, inner, dist_chunk(j), unroll=4)
                gm[pl.ds(j * 16, 16)] = acc
                return jnp.minimum(vh, acc)

            gmm[pl.ds(0, 16)] = lax.fori_loop(0, 16, outer, inf16)
            gmm[pl.ds(16, 16)] = lax.fori_loop(16, 32, outer, inf16)

            def extract(i, ks):
                m2a = gmm[pl.ds(0, 16)]
                m2b = gmm[pl.ds(16, 16)]
                gmin = vmin_splat(jnp.minimum(m2a, m2b))
                f_a = plsc.all_reduce_ffs(m2a == gmin)
                f_b = plsc.all_reduce_ffs(m2b == gmin)
                isa = f_a < sixteen16
                l2 = jnp.where(isa, f_a, f_b)
                hcell = jnp.where(isa, zero16, sixteen16)
                hbase = hcell * 16
                gmv = plsc.load_gather(gm, [iota16x + hbase + l2])
                jloc = plsc.all_reduce_ffs(gmv == gmin)
                gstar = hbase + jloc * 16 + l2
                midx = iota512 + gstar
                dv = plsc.load_gather(dbuf, [midx])
                sd, si = plsc.sort_key_val(dv, midx)
                sgd, _sgi = plsc.sort_key_val(gmv, gmv)
                plsc.store_scatter(idxout, [ks], si, mask=lane0)
                plsc.store_scatter(dbuf, [si], inf16, mask=lane0)
                plsc.store_scatter(gm, [gstar], sd, mask=lane1)
                plsc.store_scatter(gmm, [hcell + l2],
                                   jnp.minimum(sgd, sd), mask=lane1)
                return ks + 1

            lax.fori_loop(0, _K, extract, qis * _K)
            return qis + 1

        lax.fori_loop(0, _QPW, per_query, zero16)
        pltpu.sync_copy(idxout, out_hbm.at[pl.ds(w * _QPW * _K, _QPW * _K)])

    return kk(pts, qprep)


_CH = 128


def _gather_sc(table, flat_idx):
    per_w = _TOTAL // _NW
    n_ch = per_w // _CH
    info = plsc.get_sparse_core_info()
    nc = info.num_cores
    mesh = plsc.VectorSubcoreMesh(core_axis_name="c", subcore_axis_name="s")

    @functools.partial(
        pl.kernel,
        mesh=mesh,
        compiler_params=pltpu.CompilerParams(
            use_tc_tiling_on_sc=False, needs_layout_passes=False),
        out_type=jax.ShapeDtypeStruct((_TOTAL, _C), jnp.float32),
        scratch_types=[
            pltpu.VMEM((_CH,), jnp.int32),
            pltpu.VMEM((_CH, _C), jnp.float32),
            pltpu.SemaphoreType.DMA,
        ],
    )
    def gk(table_hbm, idx_hbm, out_hbm, idx_v, rows_v, sem):
        wid = lax.axis_index("s") * nc + lax.axis_index("c")

        def body(i, _):
            base = wid * per_w + i * _CH
            pltpu.sync_copy(idx_hbm.at[pl.ds(base, _CH)], idx_v)
            pltpu.async_copy(table_hbm.at[idx_v], rows_v, sem).wait()
            pltpu.sync_copy(rows_v, out_hbm.at[pl.ds(base, _CH)])
            return 0

        lax.fori_loop(0, n_ch, body, 0)

    return gk(table, flat_idx)


_SQ = _ST // _K


def _stats1_kern(v_ref, qT_ref, w1aT_ref, w1xT_ref, s1_ref, s2_ref):
    h1 = jnp.dot(v_ref[...], w1aT_ref[...], preferred_element_type=jnp.float32)
    pt = jnp.dot(qT_ref[0], w1xT_ref[...], preferred_element_type=jnp.float32)
    h1 = (h1.reshape(_SQ, _K, 64) - pt[:, None, :]).reshape(_ST, 64)

    @pl.when(pl.program_id(0) == 0)
    def _():
        s1_ref[...] = jnp.zeros_like(s1_ref)
        s2_ref[...] = jnp.zeros_like(s2_ref)

    s1_ref[...] += jnp.sum(h1, axis=0, keepdims=True)
    s2_ref[...] += jnp.sum(h1 * h1, axis=0, keepdims=True)


def _stats1(v, qT, w1aT, w1xT):
    return pl.pallas_call(
        _stats1_kern,
        grid=(_TOTAL // _ST,),
        in_specs=[
            pl.BlockSpec((_ST, _C), lambda s: (s, 0)),
            pl.BlockSpec((1, _SQ, 3), lambda s: (s // (_M // _SQ), s % (_M // _SQ), 0)),
            pl.BlockSpec((_C, 64), lambda s: (0, 0)),
            pl.BlockSpec((3, 64), lambda s: (0, 0)),
        ],
        out_specs=[
            pl.BlockSpec((1, 64), lambda s: (0, 0)),
            pl.BlockSpec((1, 64), lambda s: (0, 0)),
        ],
        out_shape=[
            jax.ShapeDtypeStruct((1, 64), jnp.float32),
            jax.ShapeDtypeStruct((1, 64), jnp.float32),
        ],
    )(v, qT, w1aT, w1xT)


def _stats2_kern(v_ref, qT_ref, w1aT_ref, w1xT_ref, s1_ref, s2_ref,
                 g1_ref, b1_ref, w2T_ref, t1_ref, t2_ref):
    h1 = jnp.dot(v_ref[...], w1aT_ref[...], preferred_element_type=jnp.float32)
    pt = jnp.dot(qT_ref[0], w1xT_ref[...], preferred_element_type=jnp.float32)
    h1 = (h1.reshape(_SQ, _K, 64) - pt[:, None, :]).reshape(_ST, 64)
    mu1 = s1_ref[...] / _TOTAL
    var1 = s2_ref[...] / _TOTAL - mu1 * mu1
    sc1 = g1_ref[...] * lax.rsqrt(var1 + _EPS)
    h1r = jnp.maximum((h1 - mu1) * sc1 + b1_ref[...], 0.0)
    h2 = jnp.dot(h1r, w2T_ref[...], preferred_element_type=jnp.float32)

    @pl.when(pl.program_id(0) == 0)
    def _():
        t1_ref[...] = jnp.zeros_like(t1_ref)
        t2_ref[...] = jnp.zeros_like(t2_ref)

    t1_ref[...] += jnp.sum(h2, axis=0, keepdims=True)
    t2_ref[...] += jnp.sum(h2 * h2, axis=0, keepdims=True)


def _stats2(v, qT, w1aT, w1xT, s1, s2, g1r, b1r, w2T):
    return pl.pallas_call(
        _stats2_kern,
        grid=(_TOTAL // _ST,),
        in_specs=[
            pl.BlockSpec((_ST, _C), lambda s: (s, 0)),
            pl.BlockSpec((1, _SQ, 3), lambda s: (s // (_M // _SQ), s % (_M // _SQ), 0)),
            pl.BlockSpec((_C, 64), lambda s: (0, 0)),
            pl.BlockSpec((3, 64), lambda s: (0, 0)),
            pl.BlockSpec((1, 64), lambda s: (0, 0)),
            pl.BlockSpec((1, 64), lambda s: (0, 0)),
            pl.BlockSpec((1, 64), lambda s: (0, 0)),
            pl.BlockSpec((1, 64), lambda s: (0, 0)),
            pl.BlockSpec((64, 128), lambda s: (0, 0)),
        ],
        out_specs=[
            pl.BlockSpec((1, 128), lambda s: (0, 0)),
            pl.BlockSpec((1, 128), lambda s: (0, 0)),
        ],
        out_shape=[
            jax.ShapeDtypeStruct((1, 128), jnp.float32),
            jax.ShapeDtypeStruct((1, 128), jnp.float32),
        ],
    )(v, qT, w1aT, w1xT, s1, s2, g1r, b1r, w2T)


def _final_kern(v_ref, qT_ref, w1aT_ref, w1xT_ref, s1_ref, s2_ref,
                g1_ref, b1_ref, w2T_ref, t1_ref, t2_ref, g2_ref, b2_ref,
                o_ref):
    v2 = v_ref[0].reshape(_MT * _K, _C)
    h1 = jnp.dot(v2, w1aT_ref[...], preferred_element_type=jnp.float32)
    pt = jnp.dot(qT_ref[0], w1xT_ref[...], preferred_element_type=jnp.float32)
    h1 = (h1.reshape(_MT, _K, 64) - pt[:, None, :]).reshape(_MT * _K, 64)
    mu1 = s1_ref[...] / _TOTAL
    var1 = s2_ref[...] / _TOTAL - mu1 * mu1
    sc1 = g1_ref[...] * lax.rsqrt(var1 + _EPS)
    h1r = jnp.maximum((h1 - mu1) * sc1 + b1_ref[...], 0.0)
    h2 = jnp.dot(h1r, w2T_ref[...], preferred_element_type=jnp.float32)
    mu2 = t1_ref[...] / _TOTAL
    var2 = t2_ref[...] / _TOTAL - mu2 * mu2
    sc2 = g2_ref[...] * lax.rsqrt(var2 + _EPS)
    h2r = jnp.maximum((h2 - mu2) * sc2 + b2_ref[...], 0.0)
    o_ref[0] = jnp.max(h2r.reshape(_MT, _K, 128), axis=1)


def _final(v4, qT, w1aT, w1xT, s1, s2, g1r, b1r, w2T, t1, t2, g2r, b2r):
    return pl.pallas_call(
        _final_kern,
        grid=(_B, _M // _MT),
        in_specs=[
            pl.BlockSpec((1, _MT, _K, _C), lambda b, t: (b, t, 0, 0)),
            pl.BlockSpec((1, _MT, 3), lambda b, t: (b, t, 0)),
            pl.BlockSpec((_C, 64), lambda b, t: (0, 0)),
            pl.BlockSpec((3, 64), lambda b, t: (0, 0)),
            pl.BlockSpec((1, 64), lambda b, t: (0, 0)),
            pl.BlockSpec((1, 64), lambda b, t: (0, 0)),
            pl.BlockSpec((1, 64), lambda b, t: (0, 0)),
            pl.BlockSpec((1, 64), lambda b, t: (0, 0)),
            pl.BlockSpec((64, 128), lambda b, t: (0, 0)),
            pl.BlockSpec((1, 128), lambda b, t: (0, 0)),
            pl.BlockSpec((1, 128), lambda b, t: (0, 0)),
            pl.BlockSpec((1, 128), lambda b, t: (0, 0)),
            pl.BlockSpec((1, 128), lambda b, t: (0, 0)),
        ],
        out_specs=pl.BlockSpec((1, _MT, 128), lambda b, t: (b, t, 0)),
        out_shape=jax.ShapeDtypeStruct((_B, _M, 128), jnp.float32),
    )(v4, qT, w1aT, w1xT, s1, s2, g1r, b1r, w2T, t1, t2, g2r, b2r)


def kernel(x, W1, g1, b1, W2, g2, b2):
    x3 = x[:, :, :, 0]
    pts = x3[:, 0:3, :]
    qc = x3[:, 0:3, ::_DS]
    qT = jnp.transpose(qc, (0, 2, 1))
    qprep = (qc.reshape(_B, 3, _NW // _B, _QPW)
             .transpose(0, 2, 1, 3).reshape(_NW, 3, _QPW))

    idx = _knn_sc(pts, qprep)

    table = jnp.transpose(x3, (0, 2, 1)).reshape(_B * _N, _C)
    flat_idx = (idx.reshape(_B, _M * _K)
                + (jnp.arange(_B, dtype=jnp.int32) * _N)[:, None]).reshape(-1)
    v = _gather_sc(table, flat_idx)

    w1a = jnp.concatenate(
        [W1[:, 0:6], jnp.zeros((64, 1), jnp.float32), W1[:, 6:15]], axis=1)
    w1aT = w1a.T
    w1xT = W1[:, 0:3].T
    g1r, b1r = g1.reshape(1, 64), b1.reshape(1, 64)
    g2r, b2r = g2.reshape(1, 128), b2.reshape(1, 128)
    w2T = W2.T

    s1, s2 = _stats1(v, qT, w1aT, w1xT)
    t1, t2 = _stats2(v, qT, w1aT, w1xT, s1, s2, g1r, b1r, w2T)
    o = _final(v.reshape(_B, _M, _K, _C), qT, w1aT, w1xT,
               s1, s2, g1r, b1r, w2T, t1, t2, g2r, b2r)

    pd = x[:, 0:_XYZN, ::_DS, :]
    return jnp.concatenate(
        [pd, jnp.transpose(o, (0, 2, 1))[..., None]], axis=1)


# speedup vs baseline: 2.3477x; 1.0965x over previous
import functools

import jax
import jax.numpy as jnp
from jax import lax
from jax.experimental import pallas as pl
from jax.experimental.pallas import tpu as pltpu
from jax.experimental.pallas import tpu_sc as plsc

_B, _C, _N = 2, 16, 8192
_DS = 4
_M = _N // _DS
_K = 32
_XYZN = 7
_EPS = 1e-5
_QT = 128
_ST = 2048
_MT = 256
_TOTAL = _B * _K * _M
_NW = 32


def _knn_kern(pts_ref, q_ref, out_ref, d_ref):
    px = pts_ref[0, 0:1, :]
    py = pts_ref[0, 1:2, :]
    pz = pts_ref[0, 2:3, :]
    qx = q_ref[0, :, 0:1]
    qy = q_ref[0, :, 1:2]
    qz = q_ref[0, :, 2:3]
    d_ref[...] = (qx - px) ** 2 + (qy - py) ** 2 + (qz - pz) ** 2
    iota = lax.broadcasted_iota(jnp.int32, (_QT, _N), 1)

    def body(k, _):
        d = d_ref[...]
        mn = jnp.min(d, axis=1, keepdims=True)
        am = jnp.min(jnp.where(d == mn, iota, _N), axis=1)
        out_ref[0, pl.ds(k, 1), :] = am[None, :]
        d_ref[...] = jnp.where(iota == am[:, None], jnp.inf, d)
        return 0

    lax.fori_loop(0, _K, body, 0)


def _knn(pts, qT):
    return pl.pallas_call(
        _knn_kern,
        grid=(_B, _M // _QT),
        in_specs=[
            pl.BlockSpec((1, 3, _N), lambda b, t: (b, 0, 0)),
            pl.BlockSpec((1, _QT, 3), lambda b, t: (b, t, 0)),
        ],
        out_specs=pl.BlockSpec((1, _K, _QT), lambda b, t: (b, 0, t)),
        out_shape=jax.ShapeDtypeStruct((_B, _K, _M), jnp.int32),
        scratch_shapes=[pltpu.VMEM((_QT, _N), jnp.float32)],
    )(pts, qT)


_QPW = _M * _B // _NW
_NG = 512
_GV = _NG // 16


def _knn_sc(pts, qprep):
    info = plsc.get_sparse_core_info()
    nc = info.num_cores
    mesh = plsc.VectorSubcoreMesh(core_axis_name="c", subcore_axis_name="s")

    @functools.partial(
        pl.kernel,
        mesh=mesh,
        compiler_params=pltpu.CompilerParams(
            use_tc_tiling_on_sc=False, needs_layout_passes=False),
        out_type=jax.ShapeDtypeStruct((_B * _M * _K,), jnp.int32),
        scratch_types=[
            pltpu.VMEM((3, _N), jnp.float32),
            pltpu.VMEM((3, _QPW), jnp.float32),
            pltpu.VMEM((2, _N), jnp.float32),
            pltpu.VMEM((2, _NG), jnp.float32),
            pltpu.VMEM((2, 32), jnp.float32),
            pltpu.VMEM((_QPW * _K,), jnp.int32),
        ],
    )
    def kk(pts_hbm, q_hbm, out_hbm, ptsv, qv, dbuf, gm, gmm, idxout):
        w = lax.axis_index("s") * nc + lax.axis_index("c")
        b = w // (_NW // _B)
        pltpu.sync_copy(pts_hbm.at[b], ptsv)
        pltpu.sync_copy(q_hbm.at[w], qv)

        iota = lax.broadcasted_iota(jnp.int32, (16,), 0)
        lane0 = iota == 0
        lane1 = iota == 1
        zero16 = jnp.zeros((16,), jnp.int32)
        one16 = jnp.full((16,), 1, jnp.int32)
        two16 = jnp.full((16,), 2, jnp.int32)
        sixteen16 = jnp.full((16,), 16, jnp.int32)
        inf16 = jnp.full((16,), jnp.inf, jnp.float32)
        iota16x = iota * 16
        iota512 = iota * _NG

        def vmin_splat(x):
            nx = -x
            return -plsc.cummax(jnp.flip(plsc.cummax(nx)))

        def per_pair(qp, qis):
            qis1 = qis + one16
            qxs0 = plsc.load_gather(qv, [zero16, qis])
            qys0 = plsc.load_gather(qv, [one16, qis])
            qzs0 = plsc.load_gather(qv, [two16, qis])
            qxs1 = plsc.load_gather(qv, [zero16, qis1])
            qys1 = plsc.load_gather(qv, [one16, qis1])
            qzs1 = plsc.load_gather(qv, [two16, qis1])

            def dist_chunk(c):
                px = ptsv[0, pl.ds(c * 16, 16)]
                py = ptsv[1, pl.ds(c * 16, 16)]
                pz = ptsv[2, pl.ds(c * 16, 16)]
                dx0 = qxs0 - px
                dy0 = qys0 - py
                dz0 = qzs0 - pz
                d0 = dx0 * dx0 + dy0 * dy0 + dz0 * dz0
                dx1 = qxs1 - px
                dy1 = qys1 - py
                dz1 = qzs1 - pz
                d1 = dx1 * dx1 + dy1 * dy1 + dz1 * dz1
                dbuf[0, pl.ds(c * 16, 16)] = d0
                dbuf[1, pl.ds(c * 16, 16)] = d1
                return d0, d1

            def outer(j, vh):
                def inner(k, acc):
                    d0, d1 = dist_chunk(j + _GV * k)
                    return jnp.minimum(acc[0], d0), jnp.minimum(acc[1], d1)

                acc0, acc1 = lax.fori_loop(1, 16, inner, dist_chunk(j),
                                           unroll=4)
                gm[0, pl.ds(j * 16, 16)] = acc0
                gm[1, pl.ds(j * 16, 16)] = acc1
                return jnp.minimum(vh[0], acc0), jnp.minimum(vh[1], acc1)

            va0, va1 = lax.fori_loop(0, 16, outer, (inf16, inf16))
            vb0, vb1 = lax.fori_loop(16, 32, outer, (inf16, inf16))
            gmm[0, pl.ds(0, 16)] = va0
            gmm[0, pl.ds(16, 16)] = vb0
            gmm[1, pl.ds(0, 16)] = va1
            gmm[1, pl.ds(16, 16)] = vb1

            def extract_one(qsel, ks):
                m2a = gmm[qsel, pl.ds(0, 16)]
                m2b = gmm[qsel, pl.ds(16, 16)]
                gmin = vmin_splat(jnp.minimum(m2a, m2b))
                f_a = plsc.all_reduce_ffs(m2a == gmin)
                f_b = plsc.all_reduce_ffs(m2b == gmin)
                isa = f_a < sixteen16
                l2 = jnp.where(isa, f_a, f_b)
                hcell = jnp.where(isa, zero16, sixteen16)
                hbase = hcell * 16
                qsel16 = jnp.full((16,), qsel, jnp.int32)
                gmv = plsc.load_gather(gm, [qsel16, iota16x + hbase + l2])
                jloc = plsc.all_reduce_ffs(gmv == gmin)
                gstar = hbase + jloc * 16 + l2
                midx = iota512 + gstar
                dv = plsc.load_gather(dbuf, [qsel16, midx])
                sd, si = plsc.sort_key_val(dv, midx)
                sgd, _sgi = plsc.sort_key_val(gmv, gmv)
                plsc.store_scatter(idxout, [ks], si, mask=lane0)
                plsc.store_scatter(dbuf, [qsel16, si], inf16, mask=lane0)
                plsc.store_scatter(gm, [qsel16, gstar], sd, mask=lane1)
                plsc.store_scatter(gmm, [qsel16, hcell + l2],
                                   jnp.minimum(sgd, sd), mask=lane1)

            def extract(i, ks):
                extract_one(0, ks)
                extract_one(1, ks + _K)
                return ks + 1

            lax.fori_loop(0, _K, extract, qis * _K)
            return qis + 2

        lax.fori_loop(0, _QPW // 2, per_pair, zero16)
        pltpu.sync_copy(idxout, out_hbm.at[pl.ds(w * _QPW * _K, _QPW * _K)])

    return kk(pts, qprep)


_CH = 128


def _gather_sc(table, flat_idx):
    per_w = _TOTAL // _NW
    n_ch = per_w // _CH
    info = plsc.get_sparse_core_info()
    nc = info.num_cores
    mesh = plsc.VectorSubcoreMesh(core_axis_name="c", subcore_axis_name="s")

    @functools.partial(
        pl.kernel,
        mesh=mesh,
        compiler_params=pltpu.CompilerParams(
            use_tc_tiling_on_sc=False, needs_layout_passes=False),
        out_type=jax.ShapeDtypeStruct((_TOTAL, _C), jnp.float32),
        scratch_types=[
            pltpu.VMEM((_CH,), jnp.int32),
            pltpu.VMEM((_CH, _C), jnp.float32),
            pltpu.SemaphoreType.DMA,
        ],
    )
    def gk(table_hbm, idx_hbm, out_hbm, idx_v, rows_v, sem):
        wid = lax.axis_index("s") * nc + lax.axis_index("c")

        def body(i, _):
            base = wid * per_w + i * _CH
            pltpu.sync_copy(idx_hbm.at[pl.ds(base, _CH)], idx_v)
            pltpu.async_copy(table_hbm.at[idx_v], rows_v, sem).wait()
            pltpu.sync_copy(rows_v, out_hbm.at[pl.ds(base, _CH)])
            return 0

        lax.fori_loop(0, n_ch, body, 0)

    return gk(table, flat_idx)


_SQ = _ST // _K


def _stats1_kern(v_ref, qT_ref, w1aT_ref, w1xT_ref, s1_ref, s2_ref):
    h1 = jnp.dot(v_ref[...], w1aT_ref[...], preferred_element_type=jnp.float32)
    pt = jnp.dot(qT_ref[0], w1xT_ref[...], preferred_element_type=jnp.float32)
    h1 = (h1.reshape(_SQ, _K, 64) - pt[:, None, :]).reshape(_ST, 64)

    @pl.when(pl.program_id(0) == 0)
    def _():
        s1_ref[...] = jnp.zeros_like(s1_ref)
        s2_ref[...] = jnp.zeros_like(s2_ref)

    s1_ref[...] += jnp.sum(h1, axis=0, keepdims=True)
    s2_ref[...] += jnp.sum(h1 * h1, axis=0, keepdims=True)


def _stats1(v, qT, w1aT, w1xT):
    return pl.pallas_call(
        _stats1_kern,
        grid=(_TOTAL // _ST,),
        in_specs=[
            pl.BlockSpec((_ST, _C), lambda s: (s, 0)),
            pl.BlockSpec((1, _SQ, 3), lambda s: (s // (_M // _SQ), s % (_M // _SQ), 0)),
            pl.BlockSpec((_C, 64), lambda s: (0, 0)),
            pl.BlockSpec((3, 64), lambda s: (0, 0)),
        ],
        out_specs=[
            pl.BlockSpec((1, 64), lambda s: (0, 0)),
            pl.BlockSpec((1, 64), lambda s: (0, 0)),
        ],
        out_shape=[
            jax.ShapeDtypeStruct((1, 64), jnp.float32),
            jax.ShapeDtypeStruct((1, 64), jnp.float32),
        ],
    )(v, qT, w1aT, w1xT)


def _stats2_kern(v_ref, qT_ref, w1aT_ref, w1xT_ref, s1_ref, s2_ref,
                 g1_ref, b1_ref, w2T_ref, t1_ref, t2_ref):
    h1 = jnp.dot(v_ref[...], w1aT_ref[...], preferred_element_type=jnp.float32)
    pt = jnp.dot(qT_ref[0], w1xT_ref[...], preferred_element_type=jnp.float32)
    h1 = (h1.reshape(_SQ, _K, 64) - pt[:, None, :]).reshape(_ST, 64)
    mu1 = s1_ref[...] / _TOTAL
    var1 = s2_ref[...] / _TOTAL - mu1 * mu1
    sc1 = g1_ref[...] * lax.rsqrt(var1 + _EPS)
    h1r = jnp.maximum((h1 - mu1) * sc1 + b1_ref[...], 0.0)
    h2 = jnp.dot(h1r, w2T_ref[...], preferred_element_type=jnp.float32)

    @pl.when(pl.program_id(0) == 0)
    def _():
        t1_ref[...] = jnp.zeros_like(t1_ref)
        t2_ref[...] = jnp.zeros_like(t2_ref)

    t1_ref[...] += jnp.sum(h2, axis=0, keepdims=True)
    t2_ref[...] += jnp.sum(h2 * h2, axis=0, keepdims=True)


def _stats2(v, qT, w1aT, w1xT, s1, s2, g1r, b1r, w2T):
    return pl.pallas_call(
        _stats2_kern,
        grid=(_TOTAL // _ST,),
        in_specs=[
            pl.BlockSpec((_ST, _C), lambda s: (s, 0)),
            pl.BlockSpec((1, _SQ, 3), lambda s: (s // (_M // _SQ), s % (_M // _SQ), 0)),
            pl.BlockSpec((_C, 64), lambda s: (0, 0)),
            pl.BlockSpec((3, 64), lambda s: (0, 0)),
            pl.BlockSpec((1, 64), lambda s: (0, 0)),
            pl.BlockSpec((1, 64), lambda s: (0, 0)),
            pl.BlockSpec((1, 64), lambda s: (0, 0)),
            pl.BlockSpec((1, 64), lambda s: (0, 0)),
            pl.BlockSpec((64, 128), lambda s: (0, 0)),
        ],
        out_specs=[
            pl.BlockSpec((1, 128), lambda s: (0, 0)),
            pl.BlockSpec((1, 128), lambda s: (0, 0)),
        ],
        out_shape=[
            jax.ShapeDtypeStruct((1, 128), jnp.float32),
            jax.ShapeDtypeStruct((1, 128), jnp.float32),
        ],
    )(v, qT, w1aT, w1xT, s1, s2, g1r, b1r, w2T)


def _final_kern(v_ref, qT_ref, w1aT_ref, w1xT_ref, s1_ref, s2_ref,
                g1_ref, b1_ref, w2T_ref, t1_ref, t2_ref, g2_ref, b2_ref,
                o_ref):
    v2 = v_ref[0].reshape(_MT * _K, _C)
    h1 = jnp.dot(v2, w1aT_ref[...], preferred_element_type=jnp.float32)
    pt = jnp.dot(qT_ref[0], w1xT_ref[...], preferred_element_type=jnp.float32)
    h1 = (h1.reshape(_MT, _K, 64) - pt[:, None, :]).reshape(_MT * _K, 64)
    mu1 = s1_ref[...] / _TOTAL
    var1 = s2_ref[...] / _TOTAL - mu1 * mu1
    sc1 = g1_ref[...] * lax.rsqrt(var1 + _EPS)
    h1r = jnp.maximum((h1 - mu1) * sc1 + b1_ref[...], 0.0)
    h2 = jnp.dot(h1r, w2T_ref[...], preferred_element_type=jnp.float32)
    mu2 = t1_ref[...] / _TOTAL
    var2 = t2_ref[...] / _TOTAL - mu2 * mu2
    sc2 = g2_ref[...] * lax.rsqrt(var2 + _EPS)
    h2r = jnp.maximum((h2 - mu2) * sc2 + b2_ref[...], 0.0)
    o_ref[0] = jnp.max(h2r.reshape(_MT, _K, 128), axis=1)


def _final(v4, qT, w1aT, w1xT, s1, s2, g1r, b1r, w2T, t1, t2, g2r, b2r):
    return pl.pallas_call(
        _final_kern,
        grid=(_B, _M // _MT),
        in_specs=[
            pl.BlockSpec((1, _MT, _K, _C), lambda b, t: (b, t, 0, 0)),
            pl.BlockSpec((1, _MT, 3), lambda b, t: (b, t, 0)),
            pl.BlockSpec((_C, 64), lambda b, t: (0, 0)),
            pl.BlockSpec((3, 64), lambda b, t: (0, 0)),
            pl.BlockSpec((1, 64), lambda b, t: (0, 0)),
            pl.BlockSpec((1, 64), lambda b, t: (0, 0)),
            pl.BlockSpec((1, 64), lambda b, t: (0, 0)),
            pl.BlockSpec((1, 64), lambda b, t: (0, 0)),
            pl.BlockSpec((64, 128), lambda b, t: (0, 0)),
            pl.BlockSpec((1, 128), lambda b, t: (0, 0)),
            pl.BlockSpec((1, 128), lambda b, t: (0, 0)),
            pl.BlockSpec((1, 128), lambda b, t: (0, 0)),
            pl.BlockSpec((1, 128), lambda b, t: (0, 0)),
        ],
        out_specs=pl.BlockSpec((1, _MT, 128), lambda b, t: (b, t, 0)),
        out_shape=jax.ShapeDtypeStruct((_B, _M, 128), jnp.float32),
    )(v4, qT, w1aT, w1xT, s1, s2, g1r, b1r, w2T, t1, t2, g2r, b2r)


def kernel(x, W1, g1, b1, W2, g2, b2):
    x3 = x[:, :, :, 0]
    pts = x3[:, 0:3, :]
    qc = x3[:, 0:3, ::_DS]
    qT = jnp.transpose(qc, (0, 2, 1))
    qprep = (qc.reshape(_B, 3, _NW // _B, _QPW)
             .transpose(0, 2, 1, 3).reshape(_NW, 3, _QPW))

    idx = _knn_sc(pts, qprep)

    table = jnp.transpose(x3, (0, 2, 1)).reshape(_B * _N, _C)
    flat_idx = (idx.reshape(_B, _M * _K)
                + (jnp.arange(_B, dtype=jnp.int32) * _N)[:, None]).reshape(-1)
    v = _gather_sc(table, flat_idx)

    w1a = jnp.concatenate(
        [W1[:, 0:6], jnp.zeros((64, 1), jnp.float32), W1[:, 6:15]], axis=1)
    w1aT = w1a.T
    w1xT = W1[:, 0:3].T
    g1r, b1r = g1.reshape(1, 64), b1.reshape(1, 64)
    g2r, b2r = g2.reshape(1, 128), b2.reshape(1, 128)
    w2T = W2.T

    s1, s2 = _stats1(v, qT, w1aT, w1xT)
    t1, t2 = _stats2(v, qT, w1aT, w1xT, s1, s2, g1r, b1r, w2T)
    o = _final(v.reshape(_B, _M, _K, _C), qT, w1aT, w1xT,
               s1, s2, g1r, b1r, w2T, t1, t2, g2r, b2r)

    pd = x[:, 0:_XYZN, ::_DS, :]
    return jnp.concatenate(
        [pd, jnp.transpose(o, (0, 2, 1))[..., None]], axis=1)


# speedup vs baseline: 2.6186x; 1.1154x over previous
import functools

import jax
import jax.numpy as jnp
from jax import lax
from jax.experimental import pallas as pl
from jax.experimental.pallas import tpu as pltpu
from jax.experimental.pallas import tpu_sc as plsc

_B, _C, _N = 2, 16, 8192
_DS = 4
_M = _N // _DS
_K = 32
_XYZN = 7
_EPS = 1e-5
_QT = 128
_ST = 4096
_MT = 256
_TOTAL = _B * _K * _M
_NW = 32


def _knn_kern(pts_ref, q_ref, out_ref, d_ref):
    px = pts_ref[0, 0:1, :]
    py = pts_ref[0, 1:2, :]
    pz = pts_ref[0, 2:3, :]
    qx = q_ref[0, :, 0:1]
    qy = q_ref[0, :, 1:2]
    qz = q_ref[0, :, 2:3]
    d_ref[...] = (qx - px) ** 2 + (qy - py) ** 2 + (qz - pz) ** 2
    iota = lax.broadcasted_iota(jnp.int32, (_QT, _N), 1)

    def body(k, _):
        d = d_ref[...]
        mn = jnp.min(d, axis=1, keepdims=True)
        am = jnp.min(jnp.where(d == mn, iota, _N), axis=1)
        out_ref[0, pl.ds(k, 1), :] = am[None, :]
        d_ref[...] = jnp.where(iota == am[:, None], jnp.inf, d)
        return 0

    lax.fori_loop(0, _K, body, 0)


def _knn(pts, qT):
    return pl.pallas_call(
        _knn_kern,
        grid=(_B, _M // _QT),
        in_specs=[
            pl.BlockSpec((1, 3, _N), lambda b, t: (b, 0, 0)),
            pl.BlockSpec((1, _QT, 3), lambda b, t: (b, t, 0)),
        ],
        out_specs=pl.BlockSpec((1, _K, _QT), lambda b, t: (b, 0, t)),
        out_shape=jax.ShapeDtypeStruct((_B, _K, _M), jnp.int32),
        scratch_shapes=[pltpu.VMEM((_QT, _N), jnp.float32)],
    )(pts, qT)


_QPW = _M * _B // _NW
_NG = 512
_GV = _NG // 16


def _knn_sc(pts, qprep):
    info = plsc.get_sparse_core_info()
    nc = info.num_cores
    mesh = plsc.VectorSubcoreMesh(core_axis_name="c", subcore_axis_name="s")

    @functools.partial(
        pl.kernel,
        mesh=mesh,
        compiler_params=pltpu.CompilerParams(
            use_tc_tiling_on_sc=False, needs_layout_passes=False),
        out_type=jax.ShapeDtypeStruct((_B * _M * _K,), jnp.int32),
        scratch_types=[
            pltpu.VMEM((3, _N), jnp.float32),
            pltpu.VMEM((3, _QPW), jnp.float32),
            pltpu.VMEM((2, _N), jnp.float32),
            pltpu.VMEM((2, _NG), jnp.float32),
            pltpu.VMEM((2, 32), jnp.float32),
            pltpu.VMEM((_QPW * _K,), jnp.int32),
        ],
    )
    def kk(pts_hbm, q_hbm, out_hbm, ptsv, qv, dbuf, gm, gmm, idxout):
        w = lax.axis_index("s") * nc + lax.axis_index("c")
        b = w // (_NW // _B)
        pltpu.sync_copy(pts_hbm.at[b], ptsv)
        pltpu.sync_copy(q_hbm.at[w], qv)

        iota = lax.broadcasted_iota(jnp.int32, (16,), 0)
        lane0 = iota == 0
        lane1 = iota == 1
        zero16 = jnp.zeros((16,), jnp.int32)
        one16 = jnp.full((16,), 1, jnp.int32)
        two16 = jnp.full((16,), 2, jnp.int32)
        sixteen16 = jnp.full((16,), 16, jnp.int32)
        inf16 = jnp.full((16,), jnp.inf, jnp.float32)
        iota16x = iota * 16
        iota512 = iota * _NG

        def vmin_splat(x):
            nx = -x
            return -plsc.cummax(jnp.flip(plsc.cummax(nx)))

        def per_pair(qp, qis):
            qis1 = qis + one16
            qxs0 = plsc.load_gather(qv, [zero16, qis])
            qys0 = plsc.load_gather(qv, [one16, qis])
            qzs0 = plsc.load_gather(qv, [two16, qis])
            qxs1 = plsc.load_gather(qv, [zero16, qis1])
            qys1 = plsc.load_gather(qv, [one16, qis1])
            qzs1 = plsc.load_gather(qv, [two16, qis1])

            def dist_chunk(c):
                px = ptsv[0, pl.ds(c * 16, 16)]
                py = ptsv[1, pl.ds(c * 16, 16)]
                pz = ptsv[2, pl.ds(c * 16, 16)]
                dx0 = qxs0 - px
                dy0 = qys0 - py
                dz0 = qzs0 - pz
                d0 = dx0 * dx0 + dy0 * dy0 + dz0 * dz0
                dx1 = qxs1 - px
                dy1 = qys1 - py
                dz1 = qzs1 - pz
                d1 = dx1 * dx1 + dy1 * dy1 + dz1 * dz1
                dbuf[0, pl.ds(c * 16, 16)] = d0
                dbuf[1, pl.ds(c * 16, 16)] = d1
                return d0, d1

            def outer(j, vh):
                def inner(k, acc):
                    d0, d1 = dist_chunk(j + _GV * k)
                    return jnp.minimum(acc[0], d0), jnp.minimum(acc[1], d1)

                acc0, acc1 = lax.fori_loop(1, 16, inner, dist_chunk(j),
                                           unroll=4)
                gm[0, pl.ds(j * 16, 16)] = acc0
                gm[1, pl.ds(j * 16, 16)] = acc1
                return jnp.minimum(vh[0], acc0), jnp.minimum(vh[1], acc1)

            va0, va1 = lax.fori_loop(0, 16, outer, (inf16, inf16))
            vb0, vb1 = lax.fori_loop(16, 32, outer, (inf16, inf16))
            gmm[0, pl.ds(0, 16)] = va0
            gmm[0, pl.ds(16, 16)] = vb0
            gmm[1, pl.ds(0, 16)] = va1
            gmm[1, pl.ds(16, 16)] = vb1

            def extract_one(qsel, ks):
                m2a = gmm[qsel, pl.ds(0, 16)]
                m2b = gmm[qsel, pl.ds(16, 16)]
                gmin = vmin_splat(jnp.minimum(m2a, m2b))
                f_a = plsc.all_reduce_ffs(m2a == gmin)
                f_b = plsc.all_reduce_ffs(m2b == gmin)
                isa = f_a < sixteen16
                l2 = jnp.where(isa, f_a, f_b)
                hcell = jnp.where(isa, zero16, sixteen16)
                hbase = hcell * 16
                qsel16 = jnp.full((16,), qsel, jnp.int32)
                gmv = plsc.load_gather(gm, [qsel16, iota16x + hbase + l2])
                jloc = plsc.all_reduce_ffs(gmv == gmin)
                gstar = hbase + jloc * 16 + l2
                midx = iota512 + gstar
                dv = plsc.load_gather(dbuf, [qsel16, midx])
                sd, si = plsc.sort_key_val(dv, midx)
                sgd, _sgi = plsc.sort_key_val(gmv, gmv)
                plsc.store_scatter(idxout, [ks], si, mask=lane0)
                plsc.store_scatter(dbuf, [qsel16, si], inf16, mask=lane0)
                plsc.store_scatter(gm, [qsel16, gstar], sd, mask=lane1)
                plsc.store_scatter(gmm, [qsel16, hcell + l2],
                                   jnp.minimum(sgd, sd), mask=lane1)

            def extract(i, ks):
                extract_one(0, ks)
                extract_one(1, ks + _K)
                return ks + 1

            lax.fori_loop(0, _K, extract, qis * _K)
            return qis + 2

        lax.fori_loop(0, _QPW // 2, per_pair, zero16)
        pltpu.sync_copy(idxout, out_hbm.at[pl.ds(w * _QPW * _K, _QPW * _K)])

    return kk(pts, qprep)


_CH = 128


def _gather_sc(table, flat_idx):
    per_w = _TOTAL // _NW
    n_ch = per_w // _CH
    info = plsc.get_sparse_core_info()
    nc = info.num_cores
    mesh = plsc.VectorSubcoreMesh(core_axis_name="c", subcore_axis_name="s")

    @functools.partial(
        pl.kernel,
        mesh=mesh,
        compiler_params=pltpu.CompilerParams(
            use_tc_tiling_on_sc=False, needs_layout_passes=False),
        out_type=jax.ShapeDtypeStruct((_TOTAL, _C), jnp.float32),
        scratch_types=[
            pltpu.VMEM((_CH,), jnp.int32),
            pltpu.VMEM((_CH, _C), jnp.float32),
            pltpu.SemaphoreType.DMA,
        ],
    )
    def gk(table_hbm, idx_hbm, out_hbm, idx_v, rows_v, sem):
        wid = lax.axis_index("s") * nc + lax.axis_index("c")

        def body(i, _):
            base = wid * per_w + i * _CH
            pltpu.sync_copy(idx_hbm.at[pl.ds(base, _CH)], idx_v)
            pltpu.async_copy(table_hbm.at[idx_v], rows_v, sem).wait()
            pltpu.sync_copy(rows_v, out_hbm.at[pl.ds(base, _CH)])
            return 0

        lax.fori_loop(0, n_ch, body, 0)

    return gk(table, flat_idx)


_SQ = _ST // _K


def _mlp_kern(v_ref, qT_ref, pd_ref, w1aT_ref, w1xT_ref, g1_ref, b1_ref,
              w2T_ref, g2_ref, b2_ref, o_ref, s1, s2, t1, t2):
    p = pl.program_id(0)
    t = pl.program_id(1)
    h1 = jnp.dot(v_ref[...], w1aT_ref[...], preferred_element_type=jnp.float32)
    pt = jnp.dot(qT_ref[0], w1xT_ref[...], preferred_element_type=jnp.float32)
    h1 = (h1.reshape(_SQ, _K, 64) - pt[:, None, :]).reshape(_ST, 64)

    @pl.when(p == 0)
    def _():
        @pl.when(t == 0)
        def _():
            s1[...] = jnp.zeros_like(s1)
            s2[...] = jnp.zeros_like(s2)

        s1[...] += jnp.sum(h1, axis=0, keepdims=True)
        s2[...] += jnp.sum(h1 * h1, axis=0, keepdims=True)

    @pl.when(p > 0)
    def _():
        mu1 = s1[...] / _TOTAL
        var1 = s2[...] / _TOTAL - mu1 * mu1
        sc1 = g1_ref[...] * lax.rsqrt(var1 + _EPS)
        h1r = jnp.maximum((h1 - mu1) * sc1 + b1_ref[...], 0.0)
        h2 = jnp.dot(h1r, w2T_ref[...], preferred_element_type=jnp.float32)

        @pl.when(p == 1)
        def _():
            @pl.when(t == 0)
            def _():
                t1[...] = jnp.zeros_like(t1)
                t2[...] = jnp.zeros_like(t2)

            t1[...] += jnp.sum(h2, axis=0, keepdims=True)
            t2[...] += jnp.sum(h2 * h2, axis=0, keepdims=True)

        @pl.when(p == 2)
        def _():
            mu2 = t1[...] / _TOTAL
            var2 = t2[...] / _TOTAL - mu2 * mu2
            sc2 = g2_ref[...] * lax.rsqrt(var2 + _EPS)
            h2r = jnp.maximum((h2 - mu2) * sc2 + b2_ref[...], 0.0)
            mx = jnp.max(h2r.reshape(_SQ, _K, 128), axis=1)
            o_ref[0, 0:7, :] = pd_ref[0]
            o_ref[0, 7:135, :] = mx.T


def _mlp(v, qT, pd, w1aT, w1xT, g1r, b1r, w2T, g2r, b2r):
    nt = _TOTAL // _ST
    spb = nt // _B
    return pl.pallas_call(
        _mlp_kern,
        grid=(3, nt),
        in_specs=[
            pl.BlockSpec((_ST, _C), lambda p, s: (s, 0)),
            pl.BlockSpec((1, _SQ, 3), lambda p, s: (s // spb, s % spb, 0)),
            pl.BlockSpec((1, 7, _SQ), lambda p, s: (s // spb, 0, s % spb)),
            pl.BlockSpec((_C, 64), lambda p, s: (0, 0)),
            pl.BlockSpec((3, 64), lambda p, s: (0, 0)),
            pl.BlockSpec((1, 64), lambda p, s: (0, 0)),
            pl.BlockSpec((1, 64), lambda p, s: (0, 0)),
            pl.BlockSpec((64, 128), lambda p, s: (0, 0)),
            pl.BlockSpec((1, 128), lambda p, s: (0, 0)),
            pl.BlockSpec((1, 128), lambda p, s: (0, 0)),
        ],
        out_specs=pl.BlockSpec(
            (1, 135, _SQ),
            lambda p, s: (jnp.where(p < 2, 0, s // spb), 0,
                          jnp.where(p < 2, 0, s % spb))),
        out_shape=jax.ShapeDtypeStruct((_B, 135, _M), jnp.float32),
        scratch_shapes=[
            pltpu.VMEM((1, 64), jnp.float32),
            pltpu.VMEM((1, 64), jnp.float32),
            pltpu.VMEM((1, 128), jnp.float32),
            pltpu.VMEM((1, 128), jnp.float32),
        ],
    )(v, qT, pd, w1aT, w1xT, g1r, b1r, w2T, g2r, b2r)


def kernel(x, W1, g1, b1, W2, g2, b2):
    x3 = x[:, :, :, 0]
    pts = x3[:, 0:3, :]
    qc = x3[:, 0:3, ::_DS]
    qT = jnp.transpose(qc, (0, 2, 1))
    qprep = (qc.reshape(_B, 3, _NW // _B, _QPW)
             .transpose(0, 2, 1, 3).reshape(_NW, 3, _QPW))

    idx = _knn_sc(pts, qprep)

    table = jnp.transpose(x3, (0, 2, 1)).reshape(_B * _N, _C)
    flat_idx = (idx.reshape(_B, _M * _K)
                + (jnp.arange(_B, dtype=jnp.int32) * _N)[:, None]).reshape(-1)
    v = _gather_sc(table, flat_idx)

    w1a = jnp.concatenate(
        [W1[:, 0:6], jnp.zeros((64, 1), jnp.float32), W1[:, 6:15]], axis=1)
    w1aT = w1a.T
    w1xT = W1[:, 0:3].T
    g1r, b1r = g1.reshape(1, 64), b1.reshape(1, 64)
    g2r, b2r = g2.reshape(1, 128), b2.reshape(1, 128)
    w2T = W2.T

    pd = x3[:, 0:_XYZN, ::_DS]
    o = _mlp(v, qT, pd, w1aT, w1xT, g1r, b1r, w2T, g2r, b2r)
    return o[..., None]


# speedup vs baseline: 2.7165x; 1.0374x over previous
import functools

import jax
import jax.numpy as jnp
from jax import lax
from jax.experimental import pallas as pl
from jax.experimental.pallas import tpu as pltpu
from jax.experimental.pallas import tpu_sc as plsc

_B, _C, _N = 2, 16, 8192
_DS = 4
_M = _N // _DS
_K = 32
_XYZN = 7
_EPS = 1e-5
_QT = 128
_ST = 4096
_MT = 256
_TOTAL = _B * _K * _M
_NW = 32


def _knn_kern(pts_ref, q_ref, out_ref, d_ref):
    px = pts_ref[0, 0:1, :]
    py = pts_ref[0, 1:2, :]
    pz = pts_ref[0, 2:3, :]
    qx = q_ref[0, :, 0:1]
    qy = q_ref[0, :, 1:2]
    qz = q_ref[0, :, 2:3]
    d_ref[...] = (qx - px) ** 2 + (qy - py) ** 2 + (qz - pz) ** 2
    iota = lax.broadcasted_iota(jnp.int32, (_QT, _N), 1)

    def body(k, _):
        d = d_ref[...]
        mn = jnp.min(d, axis=1, keepdims=True)
        am = jnp.min(jnp.where(d == mn, iota, _N), axis=1)
        out_ref[0, pl.ds(k, 1), :] = am[None, :]
        d_ref[...] = jnp.where(iota == am[:, None], jnp.inf, d)
        return 0

    lax.fori_loop(0, _K, body, 0)


def _knn(pts, qT):
    return pl.pallas_call(
        _knn_kern,
        grid=(_B, _M // _QT),
        in_specs=[
            pl.BlockSpec((1, 3, _N), lambda b, t: (b, 0, 0)),
            pl.BlockSpec((1, _QT, 3), lambda b, t: (b, t, 0)),
        ],
        out_specs=pl.BlockSpec((1, _K, _QT), lambda b, t: (b, 0, t)),
        out_shape=jax.ShapeDtypeStruct((_B, _K, _M), jnp.int32),
        scratch_shapes=[pltpu.VMEM((_QT, _N), jnp.float32)],
    )(pts, qT)


_QPW = _M * _B // _NW
_NG = 512
_GV = _NG // 16


def _knn_sc(pts, qprep):
    info = plsc.get_sparse_core_info()
    nc = info.num_cores
    mesh = plsc.VectorSubcoreMesh(core_axis_name="c", subcore_axis_name="s")

    @functools.partial(
        pl.kernel,
        mesh=mesh,
        compiler_params=pltpu.CompilerParams(
            use_tc_tiling_on_sc=False, needs_layout_passes=False),
        out_type=jax.ShapeDtypeStruct((_B * _M * _K,), jnp.int32),
        scratch_types=[
            pltpu.VMEM((3, _N), jnp.float32),
            pltpu.VMEM((3, _QPW), jnp.float32),
            pltpu.VMEM((4, _N), jnp.float32),
            pltpu.VMEM((4, _NG), jnp.float32),
            pltpu.VMEM((4, 32), jnp.float32),
            pltpu.VMEM((_QPW * _K,), jnp.int32),
        ],
    )
    def kk(pts_hbm, q_hbm, out_hbm, ptsv, qv, dbuf, gm, gmm, idxout):
        w = lax.axis_index("s") * nc + lax.axis_index("c")
        b = w // (_NW // _B)
        pltpu.sync_copy(pts_hbm.at[b], ptsv)
        pltpu.sync_copy(q_hbm.at[w], qv)

        iota = lax.broadcasted_iota(jnp.int32, (16,), 0)
        lane0 = iota == 0
        lane1 = iota == 1
        zero16 = jnp.zeros((16,), jnp.int32)
        one16 = jnp.full((16,), 1, jnp.int32)
        two16 = jnp.full((16,), 2, jnp.int32)
        sixteen16 = jnp.full((16,), 16, jnp.int32)
        inf16 = jnp.full((16,), jnp.inf, jnp.float32)
        iota16x = iota * 16
        iota512 = iota * _NG

        def vmin_splat(x):
            nx = -x
            return -plsc.cummax(jnp.flip(plsc.cummax(nx)))

        def per_quad(qp, qis):
            qs = [qis, qis + one16, qis + two16, qis + two16 + one16]
            qx = [plsc.load_gather(qv, [zero16, q]) for q in qs]
            qy = [plsc.load_gather(qv, [one16, q]) for q in qs]
            qz = [plsc.load_gather(qv, [two16, q]) for q in qs]

            def dist_chunk(c):
                px = ptsv[0, pl.ds(c * 16, 16)]
                py = ptsv[1, pl.ds(c * 16, 16)]
                pz = ptsv[2, pl.ds(c * 16, 16)]
                ds = []
                for q in range(4):
                    dx = qx[q] - px
                    dy = qy[q] - py
                    dz = qz[q] - pz
                    d = dx * dx + dy * dy + dz * dz
                    dbuf[q, pl.ds(c * 16, 16)] = d
                    ds.append(d)
                return tuple(ds)

            def outer(j, vh):
                def inner(k, acc):
                    d = dist_chunk(j + _GV * k)
                    return tuple(jnp.minimum(acc[q], d[q]) for q in range(4))

                acc = lax.fori_loop(1, 16, inner, dist_chunk(j), unroll=4)
                for q in range(4):
                    gm[q, pl.ds(j * 16, 16)] = acc[q]
                return tuple(jnp.minimum(vh[q], acc[q]) for q in range(4))

            va = lax.fori_loop(0, 16, outer, (inf16,) * 4)
            vb = lax.fori_loop(16, 32, outer, (inf16,) * 4)
            for q in range(4):
                gmm[q, pl.ds(0, 16)] = va[q]
                gmm[q, pl.ds(16, 16)] = vb[q]

            def extract_one(qsel, ks):
                m2a = gmm[qsel, pl.ds(0, 16)]
                m2b = gmm[qsel, pl.ds(16, 16)]
                gmin = vmin_splat(jnp.minimum(m2a, m2b))
                f_a = plsc.all_reduce_ffs(m2a == gmin)
                f_b = plsc.all_reduce_ffs(m2b == gmin)
                isa = f_a < sixteen16
                l2 = jnp.where(isa, f_a, f_b)
                hcell = jnp.where(isa, zero16, sixteen16)
                hbase = hcell * 16
                qsel16 = jnp.full((16,), qsel, jnp.int32)
                gmv = plsc.load_gather(gm, [qsel16, iota16x + hbase + l2])
                jloc = plsc.all_reduce_ffs(gmv == gmin)
                gstar = hbase + jloc * 16 + l2
                midx = iota512 + gstar
                dv = plsc.load_gather(dbuf, [qsel16, midx])
                sd, si = plsc.sort_key_val(dv, midx)
                sgd, _sgi = plsc.sort_key_val(gmv, gmv)
                plsc.store_scatter(idxout, [ks], si, mask=lane0)
                plsc.store_scatter(dbuf, [qsel16, si], inf16, mask=lane0)
                plsc.store_scatter(gm, [qsel16, gstar], sd, mask=lane1)
                plsc.store_scatter(gmm, [qsel16, hcell + l2],
                                   jnp.minimum(sgd, sd), mask=lane1)

            def extract(i, ks):
                for q in range(4):
                    extract_one(q, ks + q * _K)
                return ks + 1

            lax.fori_loop(0, _K, extract, qis * _K)
            return qis + 4

        lax.fori_loop(0, _QPW // 4, per_quad, zero16)
        pltpu.sync_copy(idxout, out_hbm.at[pl.ds(w * _QPW * _K, _QPW * _K)])

    return kk(pts, qprep)


_CH = 128


def _gather_sc(table, flat_idx):
    per_w = _TOTAL // _NW
    n_ch = per_w // _CH
    info = plsc.get_sparse_core_info()
    nc = info.num_cores
    mesh = plsc.VectorSubcoreMesh(core_axis_name="c", subcore_axis_name="s")

    @functools.partial(
        pl.kernel,
        mesh=mesh,
        compiler_params=pltpu.CompilerParams(
            use_tc_tiling_on_sc=False, needs_layout_passes=False),
        out_type=jax.ShapeDtypeStruct((_TOTAL, _C), jnp.float32),
        scratch_types=[
            pltpu.VMEM((_CH,), jnp.int32),
            pltpu.VMEM((_CH, _C), jnp.float32),
            pltpu.SemaphoreType.DMA,
        ],
    )
    def gk(table_hbm, idx_hbm, out_hbm, idx_v, rows_v, sem):
        wid = lax.axis_index("s") * nc + lax.axis_index("c")

        def body(i, _):
            base = wid * per_w + i * _CH
            pltpu.sync_copy(idx_hbm.at[pl.ds(base, _CH)], idx_v)
            pltpu.async_copy(table_hbm.at[idx_v], rows_v, sem).wait()
            pltpu.sync_copy(rows_v, out_hbm.at[pl.ds(base, _CH)])
            return 0

        lax.fori_loop(0, n_ch, body, 0)

    return gk(table, flat_idx)


_SQ = _ST // _K


def _mlp_kern(v_ref, qT_ref, pd_ref, w1aT_ref, w1xT_ref, g1_ref, b1_ref,
              w2T_ref, g2_ref, b2_ref, o_ref, s1, s2, t1, t2):
    p = pl.program_id(0)
    t = pl.program_id(1)
    h1 = jnp.dot(v_ref[...], w1aT_ref[...], preferred_element_type=jnp.float32)
    pt = jnp.dot(qT_ref[0], w1xT_ref[...], preferred_element_type=jnp.float32)
    h1 = (h1.reshape(_SQ, _K, 64) - pt[:, None, :]).reshape(_ST, 64)

    @pl.when(p == 0)
    def _():
        @pl.when(t == 0)
        def _():
            s1[...] = jnp.zeros_like(s1)
            s2[...] = jnp.zeros_like(s2)

        s1[...] += jnp.sum(h1, axis=0, keepdims=True)
        s2[...] += jnp.sum(h1 * h1, axis=0, keepdims=True)

    @pl.when(p > 0)
    def _():
        mu1 = s1[...] / _TOTAL
        var1 = s2[...] / _TOTAL - mu1 * mu1
        sc1 = g1_ref[...] * lax.rsqrt(var1 + _EPS)
        h1r = jnp.maximum((h1 - mu1) * sc1 + b1_ref[...], 0.0)
        h2 = jnp.dot(h1r, w2T_ref[...], preferred_element_type=jnp.float32)

        @pl.when(p == 1)
        def _():
            @pl.when(t == 0)
            def _():
                t1[...] = jnp.zeros_like(t1)
                t2[...] = jnp.zeros_like(t2)

            t1[...] += jnp.sum(h2, axis=0, keepdims=True)
            t2[...] += jnp.sum(h2 * h2, axis=0, keepdims=True)

        @pl.when(p == 2)
        def _():
            mu2 = t1[...] / _TOTAL
            var2 = t2[...] / _TOTAL - mu2 * mu2
            sc2 = g2_ref[...] * lax.rsqrt(var2 + _EPS)
            h2r = jnp.maximum((h2 - mu2) * sc2 + b2_ref[...], 0.0)
            mx = jnp.max(h2r.reshape(_SQ, _K, 128), axis=1)
            o_ref[0, 0:7, :] = pd_ref[0]
            o_ref[0, 7:135, :] = mx.T


def _mlp(v, qT, pd, w1aT, w1xT, g1r, b1r, w2T, g2r, b2r):
    nt = _TOTAL // _ST
    spb = nt // _B
    return pl.pallas_call(
        _mlp_kern,
        grid=(3, nt),
        in_specs=[
            pl.BlockSpec((_ST, _C), lambda p, s: (s, 0)),
            pl.BlockSpec((1, _SQ, 3), lambda p, s: (s // spb, s % spb, 0)),
            pl.BlockSpec((1, 7, _SQ), lambda p, s: (s // spb, 0, s % spb)),
            pl.BlockSpec((_C, 64), lambda p, s: (0, 0)),
            pl.BlockSpec((3, 64), lambda p, s: (0, 0)),
            pl.BlockSpec((1, 64), lambda p, s: (0, 0)),
            pl.BlockSpec((1, 64), lambda p, s: (0, 0)),
            pl.BlockSpec((64, 128), lambda p, s: (0, 0)),
            pl.BlockSpec((1, 128), lambda p, s: (0, 0)),
            pl.BlockSpec((1, 128), lambda p, s: (0, 0)),
        ],
        out_specs=pl.BlockSpec(
            (1, 135, _SQ),
            lambda p, s: (jnp.where(p < 2, 0, s // spb), 0,
                          jnp.where(p < 2, 0, s % spb))),
        out_shape=jax.ShapeDtypeStruct((_B, 135, _M), jnp.float32),
        scratch_shapes=[
            pltpu.VMEM((1, 64), jnp.float32),
            pltpu.VMEM((1, 64), jnp.float32),
            pltpu.VMEM((1, 128), jnp.float32),
            pltpu.VMEM((1, 128), jnp.float32),
        ],
    )(v, qT, pd, w1aT, w1xT, g1r, b1r, w2T, g2r, b2r)


def kernel(x, W1, g1, b1, W2, g2, b2):
    x3 = x[:, :, :, 0]
    pts = x3[:, 0:3, :]
    qc = x3[:, 0:3, ::_DS]
    qT = jnp.transpose(qc, (0, 2, 1))
    qprep = (qc.reshape(_B, 3, _NW // _B, _QPW)
             .transpose(0, 2, 1, 3).reshape(_NW, 3, _QPW))

    idx = _knn_sc(pts, qprep)

    table = jnp.transpose(x3, (0, 2, 1)).reshape(_B * _N, _C)
    flat_idx = (idx.reshape(_B, _M * _K)
                + (jnp.arange(_B, dtype=jnp.int32) * _N)[:, None]).reshape(-1)
    v = _gather_sc(table, flat_idx)

    w1a = jnp.concatenate(
        [W1[:, 0:6], jnp.zeros((64, 1), jnp.float32), W1[:, 6:15]], axis=1)
    w1aT = w1a.T
    w1xT = W1[:, 0:3].T
    g1r, b1r = g1.reshape(1, 64), b1.reshape(1, 64)
    g2r, b2r = g2.reshape(1, 128), b2.reshape(1, 128)
    w2T = W2.T

    pd = x3[:, 0:_XYZN, ::_DS]
    o = _mlp(v, qT, pd, w1aT, w1xT, g1r, b1r, w2T, g2r, b2r)
    return o[..., None]


# speedup vs baseline: 2.8396x; 1.0453x over previous
import functools

import jax
import jax.numpy as jnp
from jax import lax
from jax.experimental import pallas as pl
from jax.experimental.pallas import tpu as pltpu
from jax.experimental.pallas import tpu_sc as plsc

_B, _C, _N = 2, 16, 8192
_DS = 4
_M = _N // _DS
_K = 32
_XYZN = 7
_EPS = 1e-5
_QT = 128
_ST = 8192
_MT = 256
_TOTAL = _B * _K * _M
_NW = 32


def _knn_kern(pts_ref, q_ref, out_ref, d_ref):
    px = pts_ref[0, 0:1, :]
    py = pts_ref[0, 1:2, :]
    pz = pts_ref[0, 2:3, :]
    qx = q_ref[0, :, 0:1]
    qy = q_ref[0, :, 1:2]
    qz = q_ref[0, :, 2:3]
    d_ref[...] = (qx - px) ** 2 + (qy - py) ** 2 + (qz - pz) ** 2
    iota = lax.broadcasted_iota(jnp.int32, (_QT, _N), 1)

    def body(k, _):
        d = d_ref[...]
        mn = jnp.min(d, axis=1, keepdims=True)
        am = jnp.min(jnp.where(d == mn, iota, _N), axis=1)
        out_ref[0, pl.ds(k, 1), :] = am[None, :]
        d_ref[...] = jnp.where(iota == am[:, None], jnp.inf, d)
        return 0

    lax.fori_loop(0, _K, body, 0)


def _knn(pts, qT):
    return pl.pallas_call(
        _knn_kern,
        grid=(_B, _M // _QT),
        in_specs=[
            pl.BlockSpec((1, 3, _N), lambda b, t: (b, 0, 0)),
            pl.BlockSpec((1, _QT, 3), lambda b, t: (b, t, 0)),
        ],
        out_specs=pl.BlockSpec((1, _K, _QT), lambda b, t: (b, 0, t)),
        out_shape=jax.ShapeDtypeStruct((_B, _K, _M), jnp.int32),
        scratch_shapes=[pltpu.VMEM((_QT, _N), jnp.float32)],
    )(pts, qT)


_QPW = _M * _B // _NW
_NG = 512
_GV = _NG // 16


def _knn_sc(pts, qprep):
    info = plsc.get_sparse_core_info()
    nc = info.num_cores
    mesh = plsc.VectorSubcoreMesh(core_axis_name="c", subcore_axis_name="s")

    @functools.partial(
        pl.kernel,
        mesh=mesh,
        compiler_params=pltpu.CompilerParams(
            use_tc_tiling_on_sc=False, needs_layout_passes=False),
        out_type=jax.ShapeDtypeStruct((_B * _M * _K,), jnp.int32),
        scratch_types=[
            pltpu.VMEM((3, _N), jnp.float32),
            pltpu.VMEM((3, _QPW), jnp.float32),
            pltpu.VMEM((4, _N), jnp.float32),
            pltpu.VMEM((4, _NG), jnp.float32),
            pltpu.VMEM((4, 32), jnp.float32),
            pltpu.VMEM((_QPW * _K,), jnp.int32),
        ],
    )
    def kk(pts_hbm, q_hbm, out_hbm, ptsv, qv, dbuf, gm, gmm, idxout):
        w = lax.axis_index("s") * nc + lax.axis_index("c")
        b = w // (_NW // _B)
        pltpu.sync_copy(pts_hbm.at[b], ptsv)
        pltpu.sync_copy(q_hbm.at[w], qv)

        iota = lax.broadcasted_iota(jnp.int32, (16,), 0)
        lane0 = iota == 0
        lane1 = iota == 1
        zero16 = jnp.zeros((16,), jnp.int32)
        one16 = jnp.full((16,), 1, jnp.int32)
        two16 = jnp.full((16,), 2, jnp.int32)
        sixteen16 = jnp.full((16,), 16, jnp.int32)
        inf16 = jnp.full((16,), jnp.inf, jnp.float32)
        iota16x = iota * 16
        iota512 = iota * _NG

        def vmin_splat(x):
            nx = -x
            return -plsc.cummax(jnp.flip(plsc.cummax(nx)))

        def per_quad(qp, qis):
            qs = [qis, qis + one16, qis + two16, qis + two16 + one16]
            qx = [plsc.load_gather(qv, [zero16, q]) for q in qs]
            qy = [plsc.load_gather(qv, [one16, q]) for q in qs]
            qz = [plsc.load_gather(qv, [two16, q]) for q in qs]

            def dist_chunk(c):
                px = ptsv[0, pl.ds(c * 16, 16)]
                py = ptsv[1, pl.ds(c * 16, 16)]
                pz = ptsv[2, pl.ds(c * 16, 16)]
                ds = []
                for q in range(4):
                    dx = qx[q] - px
                    dy = qy[q] - py
                    dz = qz[q] - pz
                    d = dx * dx + dy * dy + dz * dz
                    dbuf[q, pl.ds(c * 16, 16)] = d
                    ds.append(d)
                return tuple(ds)

            def outer(j, vh):
                def inner(k, acc):
                    d = dist_chunk(j + _GV * k)
                    return tuple(jnp.minimum(acc[q], d[q]) for q in range(4))

                acc = lax.fori_loop(1, 16, inner, dist_chunk(j), unroll=4)
                for q in range(4):
                    gm[q, pl.ds(j * 16, 16)] = acc[q]
                return tuple(jnp.minimum(vh[q], acc[q]) for q in range(4))

            va = lax.fori_loop(0, 16, outer, (inf16,) * 4)
            vb = lax.fori_loop(16, 32, outer, (inf16,) * 4)
            for q in range(4):
                gmm[q, pl.ds(0, 16)] = va[q]
                gmm[q, pl.ds(16, 16)] = vb[q]

            def extract_one(qsel, ks):
                m2a = gmm[qsel, pl.ds(0, 16)]
                m2b = gmm[qsel, pl.ds(16, 16)]
                gmin = vmin_splat(jnp.minimum(m2a, m2b))
                f_a = plsc.all_reduce_ffs(m2a == gmin)
                f_b = plsc.all_reduce_ffs(m2b == gmin)
                isa = f_a < sixteen16
                l2 = jnp.where(isa, f_a, f_b)
                hcell = jnp.where(isa, zero16, sixteen16)
                hbase = hcell * 16
                qsel16 = jnp.full((16,), qsel, jnp.int32)
                gmv = plsc.load_gather(gm, [qsel16, iota16x + hbase + l2])
                jloc = plsc.all_reduce_ffs(gmv == gmin)
                gstar = hbase + jloc * 16 + l2
                midx = iota512 + gstar
                dv = plsc.load_gather(dbuf, [qsel16, midx])
                sd, si = plsc.sort_key_val(dv, midx)
                sgd, _sgi = plsc.sort_key_val(gmv, gmv)
                plsc.store_scatter(idxout, [ks], si, mask=lane0)
                plsc.store_scatter(dbuf, [qsel16, si], inf16, mask=lane0)
                plsc.store_scatter(gm, [qsel16, gstar], sd, mask=lane1)
                plsc.store_scatter(gmm, [qsel16, hcell + l2],
                                   jnp.minimum(sgd, sd), mask=lane1)

            def extract(i, ks):
                for q in range(4):
                    extract_one(q, ks + q * _K)
                return ks + 1

            lax.fori_loop(0, _K, extract, qis * _K)
            return qis + 4

        lax.fori_loop(0, _QPW // 4, per_quad, zero16)
        pltpu.sync_copy(idxout, out_hbm.at[pl.ds(w * _QPW * _K, _QPW * _K)])

    return kk(pts, qprep)


_CH = 128


def _gather_sc(table, flat_idx):
    per_w = _TOTAL // _NW
    n_ch = per_w // _CH
    info = plsc.get_sparse_core_info()
    nc = info.num_cores
    mesh = plsc.VectorSubcoreMesh(core_axis_name="c", subcore_axis_name="s")

    @functools.partial(
        pl.kernel,
        mesh=mesh,
        compiler_params=pltpu.CompilerParams(
            use_tc_tiling_on_sc=False, needs_layout_passes=False),
        out_type=jax.ShapeDtypeStruct((_TOTAL, _C), jnp.float32),
        scratch_types=[
            pltpu.VMEM((_CH,), jnp.int32),
            pltpu.VMEM((_CH, _C), jnp.float32),
            pltpu.SemaphoreType.DMA,
        ],
    )
    def gk(table_hbm, idx_hbm, out_hbm, idx_v, rows_v, sem):
        wid = lax.axis_index("s") * nc + lax.axis_index("c")

        def body(i, _):
            base = wid * per_w + i * _CH
            pltpu.sync_copy(idx_hbm.at[pl.ds(base, _CH)], idx_v)
            pltpu.async_copy(table_hbm.at[idx_v], rows_v, sem).wait()
            pltpu.sync_copy(rows_v, out_hbm.at[pl.ds(base, _CH)])
            return 0

        lax.fori_loop(0, n_ch, body, 0)

    return gk(table, flat_idx)


_SQ = _ST // _K


def _mlp_kern(v_ref, qT_ref, pd_ref, w1aT_ref, w1xT_ref, g1_ref, b1_ref,
              w2T_ref, g2_ref, b2_ref, o_ref, s1, s2, t1, t2):
    p = pl.program_id(0)
    t = pl.program_id(1)
    h1 = jnp.dot(v_ref[...], w1aT_ref[...], preferred_element_type=jnp.float32)
    pt = jnp.dot(qT_ref[0], w1xT_ref[...], preferred_element_type=jnp.float32)
    h1 = (h1.reshape(_SQ, _K, 64) - pt[:, None, :]).reshape(_ST, 64)

    @pl.when(p == 0)
    def _():
        @pl.when(t == 0)
        def _():
            s1[...] = jnp.zeros_like(s1)
            s2[...] = jnp.zeros_like(s2)

        s1[...] += jnp.sum(h1, axis=0, keepdims=True)
        s2[...] += jnp.sum(h1 * h1, axis=0, keepdims=True)

    @pl.when(p > 0)
    def _():
        mu1 = s1[...] / _TOTAL
        var1 = s2[...] / _TOTAL - mu1 * mu1
        sc1 = g1_ref[...] * lax.rsqrt(var1 + _EPS)
        h1r = jnp.maximum((h1 - mu1) * sc1 + b1_ref[...], 0.0)
        h2 = jnp.dot(h1r, w2T_ref[...], preferred_element_type=jnp.float32)

        @pl.when(p == 1)
        def _():
            @pl.when(t == 0)
            def _():
                t1[...] = jnp.zeros_like(t1)
                t2[...] = jnp.zeros_like(t2)

            t1[...] += jnp.sum(h2, axis=0, keepdims=True)
            t2[...] += jnp.sum(h2 * h2, axis=0, keepdims=True)

        @pl.when(p == 2)
        def _():
            mu2 = t1[...] / _TOTAL
            var2 = t2[...] / _TOTAL - mu2 * mu2
            sc2 = g2_ref[...] * lax.rsqrt(var2 + _EPS)
            h2r = jnp.maximum((h2 - mu2) * sc2 + b2_ref[...], 0.0)
            mx = jnp.max(h2r.reshape(_SQ, _K, 128), axis=1)
            o_ref[0, 0:7, :] = pd_ref[0]
            o_ref[0, 7:135, :] = mx.T


def _mlp(v, qT, pd, w1aT, w1xT, g1r, b1r, w2T, g2r, b2r):
    nt = _TOTAL // _ST
    spb = nt // _B
    return pl.pallas_call(
        _mlp_kern,
        grid=(3, nt),
        in_specs=[
            pl.BlockSpec((_ST, _C), lambda p, s: (s, 0)),
            pl.BlockSpec((1, _SQ, 3), lambda p, s: (s // spb, s % spb, 0)),
            pl.BlockSpec((1, 7, _SQ), lambda p, s: (s // spb, 0, s % spb)),
            pl.BlockSpec((_C, 64), lambda p, s: (0, 0)),
            pl.BlockSpec((3, 64), lambda p, s: (0, 0)),
            pl.BlockSpec((1, 64), lambda p, s: (0, 0)),
            pl.BlockSpec((1, 64), lambda p, s: (0, 0)),
            pl.BlockSpec((64, 128), lambda p, s: (0, 0)),
            pl.BlockSpec((1, 128), lambda p, s: (0, 0)),
            pl.BlockSpec((1, 128), lambda p, s: (0, 0)),
        ],
        out_specs=pl.BlockSpec(
            (1, 135, _SQ),
            lambda p, s: (jnp.where(p < 2, 0, s // spb), 0,
                          jnp.where(p < 2, 0, s % spb))),
        out_shape=jax.ShapeDtypeStruct((_B, 135, _M), jnp.float32),
        scratch_shapes=[
            pltpu.VMEM((1, 64), jnp.float32),
            pltpu.VMEM((1, 64), jnp.float32),
            pltpu.VMEM((1, 128), jnp.float32),
            pltpu.VMEM((1, 128), jnp.float32),
        ],
    )(v, qT, pd, w1aT, w1xT, g1r, b1r, w2T, g2r, b2r)


def kernel(x, W1, g1, b1, W2, g2, b2):
    x3 = x[:, :, :, 0]
    pts = x3[:, 0:3, :]
    qc = x3[:, 0:3, ::_DS]
    qT = jnp.transpose(qc, (0, 2, 1))
    qprep = (qc.reshape(_B, 3, _NW // _B, _QPW)
             .transpose(0, 2, 1, 3).reshape(_NW, 3, _QPW))

    idx = _knn_sc(pts, qprep)

    table = jnp.transpose(x3, (0, 2, 1)).reshape(_B * _N, _C)
    flat_idx = (idx.reshape(_B, _M * _K)
                + (jnp.arange(_B, dtype=jnp.int32) * _N)[:, None]).reshape(-1)
    v = _gather_sc(table, flat_idx)

    w1a = jnp.concatenate(
        [W1[:, 0:6], jnp.zeros((64, 1), jnp.float32), W1[:, 6:15]], axis=1)
    w1aT = w1a.T
    w1xT = W1[:, 0:3].T
    g1r, b1r = g1.reshape(1, 64), b1.reshape(1, 64)
    g2r, b2r = g2.reshape(1, 128), b2.reshape(1, 128)
    w2T = W2.T

    pd = x3[:, 0:_XYZN, ::_DS]
    o = _mlp(v, qT, pd, w1aT, w1xT, g1r, b1r, w2T, g2r, b2r)
    return o[..., None]
